# Initial kernel scaffold; baseline (speedup 1.0000x reference)
import jax
import jax.numpy as jnp
from jax.experimental import pallas as pl


def kernel(x, edge_index, W1, b1, W2, b2, Wc, bc, We, be):
    raise NotImplementedError("write your pallas kernel here")


# speedup vs baseline: 6.6774x; 6.6774x over previous
import functools

import jax
import jax.numpy as jnp
from jax import lax
from jax.experimental import pallas as pl
from jax.experimental.pallas import tpu as pltpu
from jax.experimental.pallas import tpu_sc as plsc

_N = 10000
_NP = 10240
_E = 320000
_H = 128
_C = 64
_CRF_ALPHA = 0.1
_APPNP_K = 10
_APPNP_ALPHA = 0.1

_CHUNK = 128
_RPT = _NP // 16

_NCH_G = (_E + 16 * _CHUNK - 1) // (16 * _CHUNK)
_EP_G = _NCH_G * 16 * _CHUNK
_NCH_C = (2 * _E + 16 * _CHUNK - 1) // (16 * _CHUNK)
_EP_C = _NCH_C * 16 * _CHUNK
_NCH_D = (_E + 32 * _CHUNK - 1) // (32 * _CHUNK)
_EP_D = _NCH_D * 32 * _CHUNK


def _sc_mesh():
    return plsc.VectorSubcoreMesh(
        core_axis_name="c", subcore_axis_name="s", num_cores=2, num_subcores=16)


def _make_sc_pass(fh, n_chunks, interpret=False):
    @functools.partial(
        pl.kernel,
        out_type=jax.ShapeDtypeStruct((2, _NP, fh), jnp.float32),
        mesh=_sc_mesh(),
        scratch_types=[
            pltpu.VMEM((_CHUNK,), jnp.int32),
            pltpu.VMEM((_CHUNK,), jnp.int32),
            pltpu.VMEM((_CHUNK, fh), jnp.float32),
            pltpu.VMEM_SHARED((_NP, fh), jnp.float32),
        ],
        compiler_params=pltpu.CompilerParams(use_tc_tiling_on_sc=False),
        interpret=interpret,
    )
    def pass_kernel(table, gidx, sidx, zrows, out, gi, si, rows, acc):
        cid = lax.axis_index("c")
        sid = lax.axis_index("s")
        rbase = sid * _RPT
        pltpu.sync_copy(zrows.at[pl.ds(rbase, _RPT)], acc.at[pl.ds(rbase, _RPT)])
        plsc.subcore_barrier()
        ebase = sid * (n_chunks * _CHUNK)
        off = cid * _N

        def chunk(c, carry):
            base = ebase + c * _CHUNK
            pltpu.sync_copy(gidx.at[pl.ds(base, _CHUNK)], gi)
            pltpu.sync_copy(sidx.at[pl.ds(base, _CHUNK)], si)
            for j in range(_CHUNK // 16):
                gi[pl.ds(j * 16, 16)] = gi[pl.ds(j * 16, 16)] + off
            pltpu.sync_copy(table.at[gi], rows)
            pltpu.sync_copy(rows, acc.at[si], add=True)
            return carry

        lax.fori_loop(0, n_chunks, chunk, 0)
        plsc.subcore_barrier()
        pltpu.sync_copy(acc.at[pl.ds(rbase, _RPT)], out.at[cid, pl.ds(rbase, _RPT)])

    return pass_kernel


def _make_sc_degree(interpret=False):
    @functools.partial(
        pl.kernel,
        out_type=jax.ShapeDtypeStruct((2, _NP, 16), jnp.float32),
        mesh=_sc_mesh(),
        scratch_types=[
            pltpu.VMEM((_CHUNK,), jnp.int32),
            pltpu.VMEM((_CHUNK, 16), jnp.float32),
            pltpu.VMEM_SHARED((_NP, 16), jnp.float32),
        ],
        compiler_params=pltpu.CompilerParams(use_tc_tiling_on_sc=False),
        interpret=interpret,
    )
    def deg_kernel(didx, ones_hbm, zrows, out, si, ones_v, acc):
        cid = lax.axis_index("c")
        sid = lax.axis_index("s")
        rbase = sid * _RPT
        pltpu.sync_copy(zrows.at[pl.ds(rbase, _RPT)], acc.at[pl.ds(rbase, _RPT)])
        pltpu.sync_copy(ones_hbm, ones_v)
        plsc.subcore_barrier()
        wid = cid * 16 + sid
        ebase = wid * (_NCH_D * _CHUNK)

        def chunk(c, carry):
            base = ebase + c * _CHUNK
            pltpu.sync_copy(didx.at[pl.ds(base, _CHUNK)], si)
            pltpu.sync_copy(ones_v, acc.at[si], add=True)
            return carry

        lax.fori_loop(0, _NCH_D, chunk, 0)
        plsc.subcore_barrier()
        pltpu.sync_copy(acc.at[pl.ds(rbase, _RPT)], out.at[cid, pl.ds(rbase, _RPT)])

    return deg_kernel


_R = 1000


def _tc_call(body, out_shapes, in_specs, out_specs, interpret=False):
    return pl.pallas_call(
        body,
        out_shape=out_shapes,
        grid=(_N // _R,),
        in_specs=in_specs,
        out_specs=out_specs,
        interpret=interpret,
    )


def _spec_rows(width):
    return pl.BlockSpec((_R, width), lambda i: (i, 0))


def _spec_halves(width):
    return pl.BlockSpec((2, _R, width), lambda i: (0, i, 0))


def _spec_full(a, b):
    return pl.BlockSpec((a, b), lambda i: (0, 0))


def _tc1(x, w1, degp, interpret=False):
    def body(x_ref, w_ref, d_ref, dinv_ref, t_ref):
        deg = d_ref[0, :, 0] + d_ref[1, :, 0] + 1.0
        dinv = lax.rsqrt(deg)[:, None]
        dinv_ref[...] = dinv
        y = jnp.dot(x_ref[...], w_ref[...], preferred_element_type=jnp.float32)
        t = y * dinv
        t_ref[0] = t[:, :64]
        t_ref[1] = t[:, 64:]

    return _tc_call(
        body,
        [jax.ShapeDtypeStruct((_N, 1), jnp.float32),
         jax.ShapeDtypeStruct((2, _N, 64), jnp.float32)],
        [_spec_rows(_H), _spec_full(_H, _H), _spec_halves(16)],
        [pl.BlockSpec((_R, 1), lambda i: (i, 0)), _spec_halves(64)],
        interpret,
    )(x, w1, degp)


def _tc2(agg, t0h, b1, dinv, interpret=False):
    def body(a_ref, t_ref, b_ref, dinv_ref, h_ref):
        d = dinv_ref[...]
        for c in range(2):
            v = d * (a_ref[c] + t_ref[c]) + b_ref[0, c * 64:(c + 1) * 64]
            h_ref[c] = jnp.maximum(v, 0.0)

    return _tc_call(
        body,
        jax.ShapeDtypeStruct((2, _N, 64), jnp.float32),
        [_spec_halves(64), _spec_halves(64), _spec_full(1, _H),
         pl.BlockSpec((_R, 1), lambda i: (i, 0))],
        _spec_halves(64),
        interpret,
    )(agg, t0h, b1, dinv)


def _tc3(crf, h1h, w2, dinv, interpret=False):
    def body(c_ref, h_ref, w_ref, dinv_ref, t_ref):
        h1 = jnp.concatenate([h_ref[0], h_ref[1]], axis=1)
        cr = jnp.concatenate([c_ref[0], c_ref[1]], axis=1)
        hc = (1.0 - _CRF_ALPHA) * h1 + _CRF_ALPHA * cr
        t = dinv_ref[...] * jnp.dot(hc, w_ref[...], preferred_element_type=jnp.float32)
        t_ref[0] = t[:, :64]
        t_ref[1] = t[:, 64:]

    return _tc_call(
        body,
        jax.ShapeDtypeStruct((2, _N, 64), jnp.float32),
        [_spec_halves(64), _spec_halves(64), _spec_full(_H, _H),
         pl.BlockSpec((_R, 1), lambda i: (i, 0))],
        _spec_halves(64),
        interpret,
    )(crf, h1h, w2, dinv)


def _tc4(agg, t1h, b2, wc, bc, we, be, dinv, interpret=False):
    def body(a_ref, t_ref, b2_ref, wc_ref, bc_ref, we_ref, be_ref, dinv_ref,
             ev_ref, tev_ref):
        d = dinv_ref[...]
        hs = []
        for c in range(2):
            v = d * (a_ref[c] + t_ref[c]) + b2_ref[0, c * 64:(c + 1) * 64]
            hs.append(jnp.maximum(v, 0.0))
        h2 = jnp.concatenate(hs, axis=1)
        logits = jnp.dot(h2, wc_ref[...], preferred_element_type=jnp.float32) + bc_ref[0]
        ev = jnp.maximum(
            jnp.dot(logits, we_ref[...], preferred_element_type=jnp.float32) + be_ref[0],
            0.0)
        t = d * ev
        ev_ref[0] = ev[:, :32]
        ev_ref[1] = ev[:, 32:]
        tev_ref[0] = t[:, :32]
        tev_ref[1] = t[:, 32:]

    return _tc_call(
        body,
        [jax.ShapeDtypeStruct((2, _N, 32), jnp.float32),
         jax.ShapeDtypeStruct((2, _N, 32), jnp.float32)],
        [_spec_halves(64), _spec_halves(64), _spec_full(1, _H),
         _spec_full(_H, _C), _spec_full(1, _C), _spec_full(_C, _C),
         _spec_full(1, _C), pl.BlockSpec((_R, 1), lambda i: (i, 0))],
        [_spec_halves(32), _spec_halves(32)],
        interpret,
    )(agg, t1h, b2, wc, bc, we, be, dinv)


def _tc5(agg, th, evh, dinv, interpret=False):
    def body(a_ref, t_ref, e_ref, dinv_ref, o_ref):
        d = dinv_ref[...]
        for c in range(2):
            hp = (1.0 - _APPNP_ALPHA) * d * (a_ref[c] + t_ref[c]) + _APPNP_ALPHA * e_ref[c]
            o_ref[c] = d * hp

    return _tc_call(
        body,
        jax.ShapeDtypeStruct((2, _N, 32), jnp.float32),
        [_spec_halves(32), _spec_halves(32), _spec_halves(32),
         pl.BlockSpec((_R, 1), lambda i: (i, 0))],
        _spec_halves(32),
        interpret,
    )(agg, th, evh, dinv)


def _tc6(agg, th, evh, dinv, interpret=False):
    def body(a_ref, t_ref, e_ref, dinv_ref, o_ref):
        d = dinv_ref[...]
        hs = []
        for c in range(2):
            hp = (1.0 - _APPNP_ALPHA) * d * (a_ref[c] + t_ref[c]) + _APPNP_ALPHA * e_ref[c]
            hs.append(hp)
        h = jnp.concatenate(hs, axis=1)
        m = jnp.max(h, axis=1, keepdims=True)
        z = h - m
        lse = jnp.log(jnp.sum(jnp.exp(z), axis=1, keepdims=True))
        o_ref[...] = z - lse

    return _tc_call(
        body,
        jax.ShapeDtypeStruct((_N, _C), jnp.float32),
        [_spec_halves(32), _spec_halves(32), _spec_halves(32),
         pl.BlockSpec((_R, 1), lambda i: (i, 0))],
        _spec_rows(_C),
        interpret,
    )(agg, th, evh, dinv)


def _run(x, edge_index, W1, b1, W2, b2, Wc, bc, We, be, interpret=False):
    src = edge_index[0]
    dst = edge_index[1]
    i32 = jnp.int32

    pad_g = _EP_G - _E
    gidx_g = jnp.concatenate([src, jnp.zeros((pad_g,), i32)])
    sidx_g = jnp.concatenate([dst, jnp.full((pad_g,), _N, i32)])
    pad_c = _EP_C - 2 * _E
    gidx_c = jnp.concatenate([dst, src, jnp.zeros((pad_c,), i32)])
    sidx_c = jnp.concatenate([src, dst, jnp.full((pad_c,), _N, i32)])
    pad_d = _EP_D - _E
    didx = jnp.concatenate([dst, jnp.full((pad_d,), _N, i32)])

    z64 = jnp.zeros((_NP, 64), jnp.float32)
    z32 = jnp.zeros((_NP, 32), jnp.float32)
    z16 = jnp.zeros((_NP, 16), jnp.float32)
    on16 = jnp.ones((_CHUNK, 16), jnp.float32)

    b1r = b1.reshape(1, _H)
    b2r = b2.reshape(1, _H)
    bcr = bc.reshape(1, _C)
    ber = be.reshape(1, _C)

    pass64 = _make_sc_pass(64, _NCH_G, interpret)
    pass64c = _make_sc_pass(64, _NCH_C, interpret)
    pass32 = _make_sc_pass(32, _NCH_G, interpret)

    degp = _make_sc_degree(interpret)(didx, on16, z16)
    dinv, t0h = _tc1(x, W1, degp, interpret)

    agg1 = pass64(t0h.reshape(2 * _N, 64), gidx_g, sidx_g, z64)
    h1h = _tc2(agg1, t0h, b1r, dinv, interpret)

    crf = pass64c(h1h.reshape(2 * _N, 64), gidx_c, sidx_c, z64)
    t1h = _tc3(crf, h1h, W2, dinv, interpret)

    agg2 = pass64(t1h.reshape(2 * _N, 64), gidx_g, sidx_g, z64)
    evh, th = _tc4(agg2, t1h, b2r, Wc, bcr, We, ber, dinv, interpret)

    for _ in range(_APPNP_K - 1):
        agg = pass32(th.reshape(2 * _N, 32), gidx_g, sidx_g, z32)
        th = _tc5(agg, th, evh, dinv, interpret)
    agg = pass32(th.reshape(2 * _N, 32), gidx_g, sidx_g, z32)
    return _tc6(agg, th, evh, dinv, interpret)


def kernel(x, edge_index, W1, b1, W2, b2, Wc, bc, We, be):
    return _run(x, edge_index, W1, b1, W2, b2, Wc, bc, We, be)


# speedup vs baseline: 9.6326x; 1.4426x over previous
import functools

import jax
import jax.numpy as jnp
from jax import lax
from jax.experimental import pallas as pl
from jax.experimental.pallas import tpu as pltpu
from jax.experimental.pallas import tpu_sc as plsc

_N = 10000
_NP = 10240
_E = 320000
_H = 128
_C = 64
_CRF_ALPHA = 0.1
_APPNP_K = 10
_APPNP_ALPHA = 0.1

_CHUNK = 128
_RPT = _NP // 16

_SUP = 16


def _round_chunks(n_edges, n_workers):
    per = -(-n_edges // (n_workers * _CHUNK))
    per = -(-per // (2 * _SUP)) * (2 * _SUP)
    return per


_NCH_G = _round_chunks(_E, 16)
_EP_G = _NCH_G * 16 * _CHUNK
_NCH_C = _round_chunks(2 * _E, 16)
_EP_C = _NCH_C * 16 * _CHUNK
_NCH_D = _round_chunks(_E, 32)
_EP_D = _NCH_D * 32 * _CHUNK


def _sc_mesh():
    return plsc.VectorSubcoreMesh(
        core_axis_name="c", subcore_axis_name="s", num_cores=2, num_subcores=16)


def _make_sc_pass(fh, n_chunks, interpret=False):
    nsup = n_chunks // _SUP

    @functools.partial(
        pl.kernel,
        out_type=jax.ShapeDtypeStruct((2, _NP, fh), jnp.float32),
        mesh=_sc_mesh(),
        scratch_types=[
            pltpu.VMEM((_SUP, _CHUNK), jnp.int32),
            pltpu.VMEM((_SUP, _CHUNK), jnp.int32),
            pltpu.VMEM((_SUP, _CHUNK), jnp.int32),
            pltpu.VMEM((_SUP, _CHUNK), jnp.int32),
            pltpu.VMEM((_CHUNK, fh), jnp.float32),
            pltpu.VMEM((_CHUNK, fh), jnp.float32),
            pltpu.VMEM_SHARED((_NP, fh), jnp.float32),
            pltpu.SemaphoreType.DMA,
            pltpu.SemaphoreType.DMA,
            pltpu.SemaphoreType.DMA,
            pltpu.SemaphoreType.DMA,
            pltpu.SemaphoreType.DMA,
            pltpu.SemaphoreType.DMA,
        ],
        compiler_params=pltpu.CompilerParams(use_tc_tiling_on_sc=False),
        interpret=interpret,
    )
    def pass_kernel(table, gidx3, sidx3, zrows, out,
                    gi0, gi1, si0, si1, rows0, rows1, acc,
                    sx0, sx1, sg0, sg1, ss0, ss1):
        cid = lax.axis_index("c")
        sid = lax.axis_index("s")
        gis = (gi0, gi1)
        sis = (si0, si1)
        rows = (rows0, rows1)
        sxs = (sx0, sx1)
        sgs = (sg0, sg1)
        sss = (ss0, ss1)
        rbase = sid * _RPT
        pltpu.sync_copy(zrows.at[pl.ds(rbase, _RPT)], acc.at[pl.ds(rbase, _RPT)])
        plsc.subcore_barrier()
        cb0 = sid * n_chunks

        def idx_start(s, b):
            cb = cb0 + s * _SUP
            pltpu.async_copy(gidx3.at[cid, pl.ds(cb, _SUP)], gis[b], sxs[b])
            pltpu.async_copy(sidx3.at[pl.ds(cb, _SUP)], sis[b], sxs[b])

        def idx_wait(s, b):
            cb = cb0 + s * _SUP
            pltpu.make_async_copy(gidx3.at[cid, pl.ds(cb, _SUP)], gis[b], sxs[b]).wait()
            pltpu.make_async_copy(sidx3.at[pl.ds(cb, _SUP)], sis[b], sxs[b]).wait()

        idx_start(0, 0)

        def sup_body(t, carry):
            for ph in range(2):
                s = 2 * t + ph
                idx_wait(s, ph)
                s_next = jnp.where(s + 1 == nsup, 0, s + 1)
                idx_start(s_next, 1 - ph)
                g_desc = [None] * _SUP
                s_desc = [None] * _SUP
                g_desc[0] = pltpu.async_copy(table.at[gis[ph].at[0]], rows[0], sgs[0])
                for j in range(1, _SUP):
                    b = j & 1
                    if j >= 2:
                        s_desc[j - 2].wait()
                    g_desc[j] = pltpu.async_copy(table.at[gis[ph].at[j]], rows[b], sgs[b])
                    g_desc[j - 1].wait()
                    s_desc[j - 1] = pltpu.async_copy(
                        rows[1 - b], acc.at[sis[ph].at[j - 1]], sss[1 - b], add=True)
                last = _SUP - 1
                g_desc[last].wait()
                s_desc[last] = pltpu.async_copy(
                    rows[last & 1], acc.at[sis[ph].at[last]], sss[last & 1], add=True)
                s_desc[last - 1].wait()
                s_desc[last].wait()
            return carry

        lax.fori_loop(0, nsup // 2, sup_body, 0)
        idx_wait(0, 0)
        plsc.subcore_barrier()
        pltpu.sync_copy(acc.at[pl.ds(rbase, _RPT)], out.at[cid, pl.ds(rbase, _RPT)])

    return pass_kernel


def _make_sc_degree(interpret=False):
    nsup = _NCH_D // _SUP

    @functools.partial(
        pl.kernel,
        out_type=jax.ShapeDtypeStruct((2, _NP, 16), jnp.float32),
        mesh=_sc_mesh(),
        scratch_types=[
            pltpu.VMEM((_SUP, _CHUNK), jnp.int32),
            pltpu.VMEM((_SUP, _CHUNK), jnp.int32),
            pltpu.VMEM((_CHUNK, 16), jnp.float32),
            pltpu.VMEM_SHARED((_NP, 16), jnp.float32),
            pltpu.SemaphoreType.DMA,
            pltpu.SemaphoreType.DMA,
            pltpu.SemaphoreType.DMA,
        ],
        compiler_params=pltpu.CompilerParams(use_tc_tiling_on_sc=False),
        interpret=interpret,
    )
    def deg_kernel(didx3, ones_hbm, zrows, out, si0, si1, ones_v, acc, sx0, sx1, ss):
        cid = lax.axis_index("c")
        sid = lax.axis_index("s")
        sis = (si0, si1)
        sxs = (sx0, sx1)
        rbase = sid * _RPT
        pltpu.sync_copy(zrows.at[pl.ds(rbase, _RPT)], acc.at[pl.ds(rbase, _RPT)])
        pltpu.sync_copy(ones_hbm, ones_v)
        plsc.subcore_barrier()
        wid = cid * 16 + sid
        cb0 = wid * _NCH_D

        def idx_start(s, b):
            pltpu.async_copy(didx3.at[pl.ds(cb0 + s * _SUP, _SUP)], sis[b], sxs[b])

        def idx_wait(s, b):
            pltpu.make_async_copy(
                didx3.at[pl.ds(cb0 + s * _SUP, _SUP)], sis[b], sxs[b]).wait()

        idx_start(0, 0)

        def sup_body(t, carry):
            for ph in range(2):
                s = 2 * t + ph
                idx_wait(s, ph)
                s_next = jnp.where(s + 1 == nsup, 0, s + 1)
                idx_start(s_next, 1 - ph)
                descs = [
                    pltpu.async_copy(ones_v, acc.at[sis[ph].at[j]], ss, add=True)
                    for j in range(_SUP)
                ]
                for d in descs:
                    d.wait()
            return carry

        lax.fori_loop(0, nsup // 2, sup_body, 0)
        idx_wait(0, 0)
        plsc.subcore_barrier()
        pltpu.sync_copy(acc.at[pl.ds(rbase, _RPT)], out.at[cid, pl.ds(rbase, _RPT)])

    return deg_kernel


_R = 1000


def _tc_call(body, out_shapes, in_specs, out_specs, interpret=False):
    return pl.pallas_call(
        body,
        out_shape=out_shapes,
        grid=(_N // _R,),
        in_specs=in_specs,
        out_specs=out_specs,
        interpret=interpret,
    )


def _spec_rows(width):
    return pl.BlockSpec((_R, width), lambda i: (i, 0))


def _spec_halves(width):
    return pl.BlockSpec((2, _R, width), lambda i: (0, i, 0))


def _spec_full(a, b):
    return pl.BlockSpec((a, b), lambda i: (0, 0))


def _tc1(x, w1, degp, interpret=False):
    def body(x_ref, w_ref, d_ref, dinv_ref, t_ref):
        deg = d_ref[0, :, 0] + d_ref[1, :, 0] + 1.0
        dinv = lax.rsqrt(deg)[:, None]
        dinv_ref[...] = dinv
        y = jnp.dot(x_ref[...], w_ref[...], preferred_element_type=jnp.float32)
        t = y * dinv
        t_ref[0] = t[:, :64]
        t_ref[1] = t[:, 64:]

    return _tc_call(
        body,
        [jax.ShapeDtypeStruct((_N, 1), jnp.float32),
         jax.ShapeDtypeStruct((2, _N, 64), jnp.float32)],
        [_spec_rows(_H), _spec_full(_H, _H), _spec_halves(16)],
        [pl.BlockSpec((_R, 1), lambda i: (i, 0)), _spec_halves(64)],
        interpret,
    )(x, w1, degp)


def _tc2(agg, t0h, b1, dinv, interpret=False):
    def body(a_ref, t_ref, b_ref, dinv_ref, h_ref):
        d = dinv_ref[...]
        for c in range(2):
            v = d * (a_ref[c] + t_ref[c]) + b_ref[0, c * 64:(c + 1) * 64]
            h_ref[c] = jnp.maximum(v, 0.0)

    return _tc_call(
        body,
        jax.ShapeDtypeStruct((2, _N, 64), jnp.float32),
        [_spec_halves(64), _spec_halves(64), _spec_full(1, _H),
         pl.BlockSpec((_R, 1), lambda i: (i, 0))],
        _spec_halves(64),
        interpret,
    )(agg, t0h, b1, dinv)


def _tc3(crf, h1h, w2, dinv, interpret=False):
    def body(c_ref, h_ref, w_ref, dinv_ref, t_ref):
        h1 = jnp.concatenate([h_ref[0], h_ref[1]], axis=1)
        cr = jnp.concatenate([c_ref[0], c_ref[1]], axis=1)
        hc = (1.0 - _CRF_ALPHA) * h1 + _CRF_ALPHA * cr
        t = dinv_ref[...] * jnp.dot(hc, w_ref[...], preferred_element_type=jnp.float32)
        t_ref[0] = t[:, :64]
        t_ref[1] = t[:, 64:]

    return _tc_call(
        body,
        jax.ShapeDtypeStruct((2, _N, 64), jnp.float32),
        [_spec_halves(64), _spec_halves(64), _spec_full(_H, _H),
         pl.BlockSpec((_R, 1), lambda i: (i, 0))],
        _spec_halves(64),
        interpret,
    )(crf, h1h, w2, dinv)


def _tc4(agg, t1h, b2, wc, bc, we, be, dinv, interpret=False):
    def body(a_ref, t_ref, b2_ref, wc_ref, bc_ref, we_ref, be_ref, dinv_ref,
             ev_ref, tev_ref):
        d = dinv_ref[...]
        hs = []
        for c in range(2):
            v = d * (a_ref[c] + t_ref[c]) + b2_ref[0, c * 64:(c + 1) * 64]
            hs.append(jnp.maximum(v, 0.0))
        h2 = jnp.concatenate(hs, axis=1)
        logits = jnp.dot(h2, wc_ref[...], preferred_element_type=jnp.float32) + bc_ref[0]
        ev = jnp.maximum(
            jnp.dot(logits, we_ref[...], preferred_element_type=jnp.float32) + be_ref[0],
            0.0)
        t = d * ev
        ev_ref[0] = ev[:, :32]
        ev_ref[1] = ev[:, 32:]
        tev_ref[0] = t[:, :32]
        tev_ref[1] = t[:, 32:]

    return _tc_call(
        body,
        [jax.ShapeDtypeStruct((2, _N, 32), jnp.float32),
         jax.ShapeDtypeStruct((2, _N, 32), jnp.float32)],
        [_spec_halves(64), _spec_halves(64), _spec_full(1, _H),
         _spec_full(_H, _C), _spec_full(1, _C), _spec_full(_C, _C),
         _spec_full(1, _C), pl.BlockSpec((_R, 1), lambda i: (i, 0))],
        [_spec_halves(32), _spec_halves(32)],
        interpret,
    )(agg, t1h, b2, wc, bc, we, be, dinv)


def _tc5(agg, th, evh, dinv, interpret=False):
    def body(a_ref, t_ref, e_ref, dinv_ref, o_ref):
        d = dinv_ref[...]
        for c in range(2):
            hp = (1.0 - _APPNP_ALPHA) * d * (a_ref[c] + t_ref[c]) + _APPNP_ALPHA * e_ref[c]
            o_ref[c] = d * hp

    return _tc_call(
        body,
        jax.ShapeDtypeStruct((2, _N, 32), jnp.float32),
        [_spec_halves(32), _spec_halves(32), _spec_halves(32),
         pl.BlockSpec((_R, 1), lambda i: (i, 0))],
        _spec_halves(32),
        interpret,
    )(agg, th, evh, dinv)


def _tc6(agg, th, evh, dinv, interpret=False):
    def body(a_ref, t_ref, e_ref, dinv_ref, o_ref):
        d = dinv_ref[...]
        hs = []
        for c in range(2):
            hp = (1.0 - _APPNP_ALPHA) * d * (a_ref[c] + t_ref[c]) + _APPNP_ALPHA * e_ref[c]
            hs.append(hp)
        h = jnp.concatenate(hs, axis=1)
        m = jnp.max(h, axis=1, keepdims=True)
        z = h - m
        lse = jnp.log(jnp.sum(jnp.exp(z), axis=1, keepdims=True))
        o_ref[...] = z - lse

    return _tc_call(
        body,
        jax.ShapeDtypeStruct((_N, _C), jnp.float32),
        [_spec_halves(32), _spec_halves(32), _spec_halves(32),
         pl.BlockSpec((_R, 1), lambda i: (i, 0))],
        _spec_rows(_C),
        interpret,
    )(agg, th, evh, dinv)


def _run(x, edge_index, W1, b1, W2, b2, Wc, bc, We, be, interpret=False):
    src = edge_index[0]
    dst = edge_index[1]
    i32 = jnp.int32

    pad_g = _EP_G - _E
    gidx_g = jnp.concatenate([src, jnp.zeros((pad_g,), i32)])
    sidx_g = jnp.concatenate([dst, jnp.full((pad_g,), _N, i32)])
    pad_c = _EP_C - 2 * _E
    gidx_c = jnp.concatenate([dst, src, jnp.zeros((pad_c,), i32)])
    sidx_c = jnp.concatenate([src, dst, jnp.full((pad_c,), _N, i32)])
    pad_d = _EP_D - _E
    didx = jnp.concatenate([dst, jnp.full((pad_d,), _N, i32)])

    gidx_g = jnp.stack([gidx_g, gidx_g + _N]).reshape(2, _EP_G // _CHUNK, _CHUNK)
    sidx_g = sidx_g.reshape(_EP_G // _CHUNK, _CHUNK)
    gidx_c = jnp.stack([gidx_c, gidx_c + _N]).reshape(2, _EP_C // _CHUNK, _CHUNK)
    sidx_c = sidx_c.reshape(_EP_C // _CHUNK, _CHUNK)
    didx = didx.reshape(_EP_D // _CHUNK, _CHUNK)

    z64 = jnp.zeros((_NP, 64), jnp.float32)
    z32 = jnp.zeros((_NP, 32), jnp.float32)
    z16 = jnp.zeros((_NP, 16), jnp.float32)
    on16 = jnp.ones((_CHUNK, 16), jnp.float32)

    b1r = b1.reshape(1, _H)
    b2r = b2.reshape(1, _H)
    bcr = bc.reshape(1, _C)
    ber = be.reshape(1, _C)

    pass64 = _make_sc_pass(64, _NCH_G, interpret)
    pass64c = _make_sc_pass(64, _NCH_C, interpret)
    pass32 = _make_sc_pass(32, _NCH_G, interpret)

    degp = _make_sc_degree(interpret)(didx, on16, z16)
    dinv, t0h = _tc1(x, W1, degp, interpret)

    agg1 = pass64(t0h.reshape(2 * _N, 64), gidx_g, sidx_g, z64)
    h1h = _tc2(agg1, t0h, b1r, dinv, interpret)

    crf = pass64c(h1h.reshape(2 * _N, 64), gidx_c, sidx_c, z64)
    t1h = _tc3(crf, h1h, W2, dinv, interpret)

    agg2 = pass64(t1h.reshape(2 * _N, 64), gidx_g, sidx_g, z64)
    evh, th = _tc4(agg2, t1h, b2r, Wc, bcr, We, ber, dinv, interpret)

    for _ in range(_APPNP_K - 1):
        agg = pass32(th.reshape(2 * _N, 32), gidx_g, sidx_g, z32)
        th = _tc5(agg, th, evh, dinv, interpret)
    agg = pass32(th.reshape(2 * _N, 32), gidx_g, sidx_g, z32)
    return _tc6(agg, th, evh, dinv, interpret)


def kernel(x, edge_index, W1, b1, W2, b2, Wc, bc, We, be):
    return _run(x, edge_index, W1, b1, W2, b2, Wc, bc, We, be)


# speedup vs baseline: 9.9005x; 1.0278x over previous
import functools

import jax
import jax.numpy as jnp
from jax import lax
from jax.experimental import pallas as pl
from jax.experimental.pallas import tpu as pltpu
from jax.experimental.pallas import tpu_sc as plsc

_N = 10000
_NP = 10240
_E = 320000
_H = 128
_C = 64
_CRF_ALPHA = 0.1
_APPNP_K = 10
_APPNP_ALPHA = 0.1

_CHUNK = 128
_RPT = _NP // 16

_SUP = 16


def _round_chunks(n_edges, n_workers):
    per = -(-n_edges // (n_workers * _CHUNK))
    per = -(-per // (2 * _SUP)) * (2 * _SUP)
    return per


_NCH_G = _round_chunks(_E, 16)
_EP_G = _NCH_G * 16 * _CHUNK
_NCH_C = _round_chunks(2 * _E, 16)
_EP_C = _NCH_C * 16 * _CHUNK
_NCH_D = _round_chunks(_E, 32)
_EP_D = _NCH_D * 32 * _CHUNK


def _sc_mesh():
    return plsc.VectorSubcoreMesh(
        core_axis_name="c", subcore_axis_name="s", num_cores=2, num_subcores=16)


def _edge_pipeline(table, gidx3, sidx3, gib, sib, rowsb, acc,
                   sxs, sgs, sss, n_chunks, cb0, cid):
    nsup = n_chunks // _SUP

    def sup_body(t, carry):
        for ph in range(2):
            s = 2 * t + ph
            _idx_wait(gidx3, sidx3, gib, sib, sxs, cb0, s, ph, cid)
            s_next = jnp.where(s + 1 == nsup, 0, s + 1)
            _idx_start(gidx3, sidx3, gib, sib, sxs, cb0, s_next, 1 - ph, cid)
            gd = [None] * _SUP
            sd = [None] * _SUP
            for j in range(_SUP):
                b = j % 4
                if j >= 4:
                    sd[j - 4].wait()
                gd[j] = pltpu.async_copy(
                    table.at[gib.at[ph, j]], rowsb.at[b], sgs[b])
                if j >= 1:
                    gd[j - 1].wait()
                    sd[j - 1] = pltpu.async_copy(
                        rowsb.at[(j - 1) % 4], acc.at[sib.at[ph, j - 1]],
                        sss[(j - 1) % 4], add=True)
            last = _SUP - 1
            gd[last].wait()
            sd[last] = pltpu.async_copy(
                rowsb.at[last % 4], acc.at[sib.at[ph, last]], sss[last % 4],
                add=True)
            for j in range(_SUP - 4, _SUP):
                sd[j].wait()
        return carry

    lax.fori_loop(0, nsup // 2, sup_body, 0)


def _idx_start(gidx3, sidx3, gib, sib, sxs, cb0, s, b, cid):
    cb = cb0 + s * _SUP
    pltpu.async_copy(gidx3.at[cid, pl.ds(cb, _SUP)], gib.at[b], sxs[b])
    pltpu.async_copy(sidx3.at[pl.ds(cb, _SUP)], sib.at[b], sxs[b])


def _idx_wait(gidx3, sidx3, gib, sib, sxs, cb0, s, b, cid):
    cb = cb0 + s * _SUP
    pltpu.make_async_copy(gidx3.at[cid, pl.ds(cb, _SUP)], gib.at[b], sxs[b]).wait()
    pltpu.make_async_copy(sidx3.at[pl.ds(cb, _SUP)], sib.at[b], sxs[b]).wait()


def _make_sc_pass(fh, n_chunks, interpret=False):
    @functools.partial(
        pl.kernel,
        out_type=jax.ShapeDtypeStruct((2, _NP, fh), jnp.float32),
        mesh=_sc_mesh(),
        scratch_types=[
            pltpu.VMEM((2, _SUP, _CHUNK), jnp.int32),
            pltpu.VMEM((2, _SUP, _CHUNK), jnp.int32),
            pltpu.VMEM((4, _CHUNK, fh), jnp.float32),
            pltpu.VMEM_SHARED((_NP, fh), jnp.float32),
            pltpu.SemaphoreType.DMA,
            pltpu.SemaphoreType.DMA,
            pltpu.SemaphoreType.DMA,
            pltpu.SemaphoreType.DMA,
            pltpu.SemaphoreType.DMA,
            pltpu.SemaphoreType.DMA,
            pltpu.SemaphoreType.DMA,
            pltpu.SemaphoreType.DMA,
            pltpu.SemaphoreType.DMA,
            pltpu.SemaphoreType.DMA,
        ],
        compiler_params=pltpu.CompilerParams(use_tc_tiling_on_sc=False),
        interpret=interpret,
    )
    def pass_kernel(table, gidx3, sidx3, zrows, out,
                    gib, sib, rowsb, acc,
                    sx0, sx1, sg0, sg1, sg2, sg3, ss0, ss1, ss2, ss3):
        cid = lax.axis_index("c")
        sid = lax.axis_index("s")
        sxs = (sx0, sx1)
        sgs = (sg0, sg1, sg2, sg3)
        sss = (ss0, ss1, ss2, ss3)
        rbase = sid * _RPT
        cb0 = sid * n_chunks
        _idx_start(gidx3, sidx3, gib, sib, sxs, cb0, 0, 0, cid)
        pltpu.sync_copy(zrows.at[pl.ds(rbase, _RPT)], acc.at[pl.ds(rbase, _RPT)])
        plsc.subcore_barrier()
        _edge_pipeline(table, gidx3, sidx3, gib, sib, rowsb, acc,
                       sxs, sgs, sss, n_chunks, cb0, cid)
        _idx_wait(gidx3, sidx3, gib, sib, sxs, cb0, 0, 0, cid)
        plsc.subcore_barrier()
        pltpu.sync_copy(acc.at[pl.ds(rbase, _RPT)], out.at[cid, pl.ds(rbase, _RPT)])

    return pass_kernel


def _make_sc_appnp(interpret=False):
    fh = 32
    nch = _NCH_G
    nblk = _RPT // _CHUNK

    @functools.partial(
        pl.kernel,
        out_type=[jax.ShapeDtypeStruct((2, _NP, fh), jnp.float32),
                  jax.ShapeDtypeStruct((2 * _NP, fh), jnp.float32)],
        mesh=_sc_mesh(),
        scratch_types=[
            pltpu.VMEM((2, _SUP, _CHUNK), jnp.int32),
            pltpu.VMEM((2, _SUP, _CHUNK), jnp.int32),
            pltpu.VMEM((4, _CHUNK, fh), jnp.float32),
            pltpu.VMEM((_RPT, fh), jnp.float32),
            pltpu.VMEM((_RPT, fh), jnp.float32),
            pltpu.VMEM((_RPT, 16), jnp.float32),
            pltpu.VMEM((_CHUNK, fh), jnp.float32),
            pltpu.VMEM((_CHUNK, fh), jnp.float32),
            pltpu.VMEM_SHARED((_NP, fh), jnp.float32),
            pltpu.SemaphoreType.DMA,
            pltpu.SemaphoreType.DMA,
            pltpu.SemaphoreType.DMA,
            pltpu.SemaphoreType.DMA,
            pltpu.SemaphoreType.DMA,
            pltpu.SemaphoreType.DMA,
            pltpu.SemaphoreType.DMA,
            pltpu.SemaphoreType.DMA,
            pltpu.SemaphoreType.DMA,
            pltpu.SemaphoreType.DMA,
        ],
        compiler_params=pltpu.CompilerParams(use_tc_tiling_on_sc=False),
        interpret=interpret,
    )
    def appnp_kernel(th, evs, dv16, gidx3, sidx3, zrows, outT, tscr,
                     gib, sib, rowsb, tbuf, evsbuf, dvbuf, abuf, zbuf, acc,
                     sx0, sx1, sg0, sg1, sg2, sg3, ss0, ss1, ss2, ss3):
        cid = lax.axis_index("c")
        sid = lax.axis_index("s")
        sxs = (sx0, sx1)
        sgs = (sg0, sg1, sg2, sg3)
        sss = (ss0, ss1, ss2, ss3)
        rbase = sid * _RPT
        cb0 = sid * nch
        tb = cid * _NP + rbase
        _idx_start(gidx3, sidx3, gib, sib, sxs, cb0, 0, 0, cid)
        pltpu.sync_copy(zrows.at[pl.ds(rbase, _RPT)], acc.at[pl.ds(rbase, _RPT)])
        pltpu.sync_copy(th.at[cid, pl.ds(rbase, _RPT)], tbuf)
        pltpu.sync_copy(evs.at[cid, pl.ds(rbase, _RPT)], evsbuf)
        pltpu.sync_copy(dv16.at[pl.ds(rbase, _RPT)], dvbuf)
        pltpu.sync_copy(zrows.at[pl.ds(0, _CHUNK)], zbuf)
        pltpu.sync_copy(tbuf, tscr.at[pl.ds(tb, _RPT)])

        def iter_body(k, carry):
            plsc.subcore_barrier()
            _edge_pipeline(tscr, gidx3, sidx3, gib, sib, rowsb, acc,
                           sxs, sgs, sss, nch, cb0, cid)
            plsc.subcore_barrier()
            for blk in range(nblk):
                row0 = rbase + blk * _CHUNK
                pltpu.sync_copy(acc.at[pl.ds(row0, _CHUNK)], abuf)

                def grp_body(g, c2, _blk=blk):
                    for u in range(16):
                        la = g * 16 + u
                        lr = _blk * _CHUNK + la
                        dv = dvbuf[lr, :]
                        for c in range(fh // 16):
                            sl = pl.ds(c * 16, 16)
                            tbuf[lr, sl] = dv * (abuf[la, sl] + tbuf[lr, sl]) + evsbuf[lr, sl]
                    return c2

                lax.fori_loop(0, _CHUNK // 16, grp_body, 0)
                pltpu.sync_copy(zbuf, acc.at[pl.ds(row0, _CHUNK)])
            pltpu.sync_copy(tbuf, tscr.at[pl.ds(tb, _RPT)])
            return carry

        lax.fori_loop(0, _APPNP_K, iter_body, 0)
        _idx_wait(gidx3, sidx3, gib, sib, sxs, cb0, 0, 0, cid)
        pltpu.sync_copy(tbuf, outT.at[cid, pl.ds(rbase, _RPT)])

    return appnp_kernel


def _make_sc_degree(interpret=False):
    nsup = _NCH_D // _SUP

    @functools.partial(
        pl.kernel,
        out_type=jax.ShapeDtypeStruct((2, _NP, 16), jnp.float32),
        mesh=_sc_mesh(),
        scratch_types=[
            pltpu.VMEM((_SUP, _CHUNK), jnp.int32),
            pltpu.VMEM((_SUP, _CHUNK), jnp.int32),
            pltpu.VMEM((_CHUNK, 16), jnp.float32),
            pltpu.VMEM_SHARED((_NP, 16), jnp.float32),
            pltpu.SemaphoreType.DMA,
            pltpu.SemaphoreType.DMA,
            pltpu.SemaphoreType.DMA,
        ],
        compiler_params=pltpu.CompilerParams(use_tc_tiling_on_sc=False),
        interpret=interpret,
    )
    def deg_kernel(didx3, ones_hbm, zrows, out, si0, si1, ones_v, acc, sx0, sx1, ss):
        cid = lax.axis_index("c")
        sid = lax.axis_index("s")
        sis = (si0, si1)
        sxs = (sx0, sx1)
        rbase = sid * _RPT
        pltpu.sync_copy(zrows.at[pl.ds(rbase, _RPT)], acc.at[pl.ds(rbase, _RPT)])
        pltpu.sync_copy(ones_hbm, ones_v)
        plsc.subcore_barrier()
        wid = cid * 16 + sid
        cb0 = wid * _NCH_D

        def idx_start(s, b):
            pltpu.async_copy(didx3.at[pl.ds(cb0 + s * _SUP, _SUP)], sis[b], sxs[b])

        def idx_wait(s, b):
            pltpu.make_async_copy(
                didx3.at[pl.ds(cb0 + s * _SUP, _SUP)], sis[b], sxs[b]).wait()

        idx_start(0, 0)

        def sup_body(t, carry):
            for ph in range(2):
                s = 2 * t + ph
                idx_wait(s, ph)
                s_next = jnp.where(s + 1 == nsup, 0, s + 1)
                idx_start(s_next, 1 - ph)
                descs = [
                    pltpu.async_copy(ones_v, acc.at[sis[ph].at[j]], ss, add=True)
                    for j in range(_SUP)
                ]
                for d in descs:
                    d.wait()
            return carry

        lax.fori_loop(0, nsup // 2, sup_body, 0)
        idx_wait(0, 0)
        plsc.subcore_barrier()
        pltpu.sync_copy(acc.at[pl.ds(rbase, _RPT)], out.at[cid, pl.ds(rbase, _RPT)])

    return deg_kernel


_R = 1000


def _tc_call(body, out_shapes, in_specs, out_specs, interpret=False):
    return pl.pallas_call(
        body,
        out_shape=out_shapes,
        grid=(_N // _R,),
        in_specs=in_specs,
        out_specs=out_specs,
        interpret=interpret,
    )


def _spec_rows(width):
    return pl.BlockSpec((_R, width), lambda i: (i, 0))


def _spec_halves(width):
    return pl.BlockSpec((2, _R, width), lambda i: (0, i, 0))


def _spec_full(a, b):
    return pl.BlockSpec((a, b), lambda i: (0, 0))


def _tc1(x, w1, degp, interpret=False):
    def body(x_ref, w_ref, d_ref, dinv_ref, t_ref):
        deg = d_ref[0, :, 0] + d_ref[1, :, 0] + 1.0
        dinv = lax.rsqrt(deg)[:, None]
        dinv_ref[...] = dinv
        y = jnp.dot(x_ref[...], w_ref[...], preferred_element_type=jnp.float32)
        t = y * dinv
        t_ref[0] = t[:, :64]
        t_ref[1] = t[:, 64:]

    return _tc_call(
        body,
        [jax.ShapeDtypeStruct((_N, 1), jnp.float32),
         jax.ShapeDtypeStruct((2, _N, 64), jnp.float32)],
        [_spec_rows(_H), _spec_full(_H, _H), _spec_halves(16)],
        [pl.BlockSpec((_R, 1), lambda i: (i, 0)), _spec_halves(64)],
        interpret,
    )(x, w1, degp)


def _tc2(agg, t0h, b1, dinv, interpret=False):
    def body(a_ref, t_ref, b_ref, dinv_ref, h_ref):
        d = dinv_ref[...]
        for c in range(2):
            v = d * (a_ref[c] + t_ref[c]) + b_ref[0, c * 64:(c + 1) * 64]
            h_ref[c] = jnp.maximum(v, 0.0)

    return _tc_call(
        body,
        jax.ShapeDtypeStruct((2, _N, 64), jnp.float32),
        [_spec_halves(64), _spec_halves(64), _spec_full(1, _H),
         pl.BlockSpec((_R, 1), lambda i: (i, 0))],
        _spec_halves(64),
        interpret,
    )(agg, t0h, b1, dinv)


def _tc3(crf, h1h, w2, dinv, interpret=False):
    def body(c_ref, h_ref, w_ref, dinv_ref, t_ref):
        h1 = jnp.concatenate([h_ref[0], h_ref[1]], axis=1)
        cr = jnp.concatenate([c_ref[0], c_ref[1]], axis=1)
        hc = (1.0 - _CRF_ALPHA) * h1 + _CRF_ALPHA * cr
        t = dinv_ref[...] * jnp.dot(hc, w_ref[...], preferred_element_type=jnp.float32)
        t_ref[0] = t[:, :64]
        t_ref[1] = t[:, 64:]

    return _tc_call(
        body,
        jax.ShapeDtypeStruct((2, _N, 64), jnp.float32),
        [_spec_halves(64), _spec_halves(64), _spec_full(_H, _H),
         pl.BlockSpec((_R, 1), lambda i: (i, 0))],
        _spec_halves(64),
        interpret,
    )(crf, h1h, w2, dinv)


def _tc4(agg, t1h, b2, wc, bc, we, be, dinv, interpret=False):
    def body(a_ref, t_ref, b2_ref, wc_ref, bc_ref, we_ref, be_ref, dinv_ref,
             th_ref, evs_ref, dv_ref):
        d = dinv_ref[...]
        hs = []
        for c in range(2):
            v = d * (a_ref[c] + t_ref[c]) + b2_ref[0, c * 64:(c + 1) * 64]
            hs.append(jnp.maximum(v, 0.0))
        h2 = jnp.concatenate(hs, axis=1)
        logits = jnp.dot(h2, wc_ref[...], preferred_element_type=jnp.float32) + bc_ref[0]
        ev = jnp.maximum(
            jnp.dot(logits, we_ref[...], preferred_element_type=jnp.float32) + be_ref[0],
            0.0)
        t = d * ev
        th_ref[0] = t[:, :32]
        th_ref[1] = t[:, 32:]
        evs = _APPNP_ALPHA * t
        evs_ref[0] = evs[:, :32]
        evs_ref[1] = evs[:, 32:]
        dv_ref[...] = jnp.broadcast_to((1.0 - _APPNP_ALPHA) * d * d, (_R, 16))

    return _tc_call(
        body,
        [jax.ShapeDtypeStruct((2, _NP, 32), jnp.float32),
         jax.ShapeDtypeStruct((2, _NP, 32), jnp.float32),
         jax.ShapeDtypeStruct((_NP, 16), jnp.float32)],
        [_spec_halves(64), _spec_halves(64), _spec_full(1, _H),
         _spec_full(_H, _C), _spec_full(1, _C), _spec_full(_C, _C),
         _spec_full(1, _C), pl.BlockSpec((_R, 1), lambda i: (i, 0))],
        [_spec_halves(32), _spec_halves(32), pl.BlockSpec((_R, 16), lambda i: (i, 0))],
        interpret,
    )(agg, t1h, b2, wc, bc, we, be, dinv)


def _tc6(tk, dinv, interpret=False):
    def body(t_ref, dinv_ref, o_ref):
        d = dinv_ref[...]
        h = jnp.concatenate([t_ref[0], t_ref[1]], axis=1) / d
        m = jnp.max(h, axis=1, keepdims=True)
        z = h - m
        lse = jnp.log(jnp.sum(jnp.exp(z), axis=1, keepdims=True))
        o_ref[...] = z - lse

    return _tc_call(
        body,
        jax.ShapeDtypeStruct((_N, _C), jnp.float32),
        [_spec_halves(32), pl.BlockSpec((_R, 1), lambda i: (i, 0))],
        _spec_rows(_C),
        interpret,
    )(tk, dinv)


def _run(x, edge_index, W1, b1, W2, b2, Wc, bc, We, be, interpret=False):
    src = edge_index[0]
    dst = edge_index[1]
    i32 = jnp.int32

    pad_g = _EP_G - _E
    gidx_g = jnp.concatenate([src, jnp.zeros((pad_g,), i32)])
    sidx_g = jnp.concatenate([dst, jnp.full((pad_g,), _N, i32)])
    pad_c = _EP_C - 2 * _E
    gidx_c = jnp.concatenate([dst, src, jnp.zeros((pad_c,), i32)])
    sidx_c = jnp.concatenate([src, dst, jnp.full((pad_c,), _N, i32)])
    pad_d = _EP_D - _E
    didx = jnp.concatenate([dst, jnp.full((pad_d,), _N, i32)])

    gidx_a = jnp.stack([gidx_g, gidx_g + _NP]).reshape(2, _EP_G // _CHUNK, _CHUNK)
    gidx_g = jnp.stack([gidx_g, gidx_g + _N]).reshape(2, _EP_G // _CHUNK, _CHUNK)
    sidx_g = sidx_g.reshape(_EP_G // _CHUNK, _CHUNK)
    gidx_c = jnp.stack([gidx_c, gidx_c + _N]).reshape(2, _EP_C // _CHUNK, _CHUNK)
    sidx_c = sidx_c.reshape(_EP_C // _CHUNK, _CHUNK)
    didx = didx.reshape(_EP_D // _CHUNK, _CHUNK)

    z64 = jnp.zeros((_NP, 64), jnp.float32)
    z32 = jnp.zeros((_NP, 32), jnp.float32)
    z16 = jnp.zeros((_NP, 16), jnp.float32)
    on16 = jnp.ones((_CHUNK, 16), jnp.float32)

    b1r = b1.reshape(1, _H)
    b2r = b2.reshape(1, _H)
    bcr = bc.reshape(1, _C)
    ber = be.reshape(1, _C)

    pass64 = _make_sc_pass(64, _NCH_G, interpret)
    pass64c = _make_sc_pass(64, _NCH_C, interpret)

    degp = _make_sc_degree(interpret)(didx, on16, z16)
    dinv, t0h = _tc1(x, W1, degp, interpret)

    agg1 = pass64(t0h.reshape(2 * _N, 64), gidx_g, sidx_g, z64)
    h1h = _tc2(agg1, t0h, b1r, dinv, interpret)

    crf = pass64c(h1h.reshape(2 * _N, 64), gidx_c, sidx_c, z64)
    t1h = _tc3(crf, h1h, W2, dinv, interpret)

    agg2 = pass64(t1h.reshape(2 * _N, 64), gidx_g, sidx_g, z64)
    th, evs, dv16 = _tc4(agg2, t1h, b2r, Wc, bcr, We, ber, dinv, interpret)

    tk, _ = _make_sc_appnp(interpret)(th, evs, dv16, gidx_a, sidx_g, z32)
    return _tc6(tk, dinv, interpret)


def kernel(x, edge_index, W1, b1, W2, b2, Wc, bc, We, be):
    return _run(x, edge_index, W1, b1, W2, b2, Wc, bc, We, be)


# speedup vs baseline: 10.3443x; 1.0448x over previous
import functools

import jax
import jax.numpy as jnp
from jax import lax
from jax.experimental import pallas as pl
from jax.experimental.pallas import tpu as pltpu
from jax.experimental.pallas import tpu_sc as plsc

_N = 10000
_NP = 10240
_E = 320000
_H = 128
_C = 64
_CRF_ALPHA = 0.1
_APPNP_K = 10
_APPNP_ALPHA = 0.1

_CHUNK = 128
_RPT = _NP // 16

_SUP = 40


def _round_chunks(n_edges, n_workers):
    per = -(-n_edges // (n_workers * _CHUNK))
    per = -(-per // (2 * _SUP)) * (2 * _SUP)
    return per


_NCH_G = _round_chunks(_E, 16)
_EP_G = _NCH_G * 16 * _CHUNK
_NCH_C = _round_chunks(2 * _E, 16)
_EP_C = _NCH_C * 16 * _CHUNK
_NCH_D = _round_chunks(_E, 32)
_EP_D = _NCH_D * 32 * _CHUNK


def _sc_mesh():
    return plsc.VectorSubcoreMesh(
        core_axis_name="c", subcore_axis_name="s", num_cores=2, num_subcores=16)


def _edge_pipeline(table, gidx3, sidx3, gib, sib, rowsb, acc,
                   sxs, sgs, sss, n_chunks, cb0, cid):
    nsup = n_chunks // _SUP

    def sup_body(t, carry):
        for ph in range(2):
            s = 2 * t + ph
            _idx_wait(gidx3, sidx3, gib, sib, sxs, cb0, s, ph, cid)
            s_next = jnp.where(s + 1 == nsup, 0, s + 1)
            _idx_start(gidx3, sidx3, gib, sib, sxs, cb0, s_next, 1 - ph, cid)
            gd = [None] * _SUP
            sd = [None] * _SUP

            def scat(j):
                gd[j].wait()
                sd[j] = pltpu.async_copy(
                    rowsb.at[j % 4], acc.at[sib.at[ph, j]], sss[j % 4], add=True)

            for j in range(_SUP):
                b = j % 4
                if j >= 4:
                    sd[j - 4].wait()
                gd[j] = pltpu.async_copy(
                    table.at[gib.at[ph, j]], rowsb.at[b], sgs[b])
                if j >= 2:
                    scat(j - 2)
            scat(_SUP - 2)
            scat(_SUP - 1)
            for j in range(_SUP - 4, _SUP):
                sd[j].wait()
        return carry

    lax.fori_loop(0, nsup // 2, sup_body, 0)


def _idx_start(gidx3, sidx3, gib, sib, sxs, cb0, s, b, cid):
    cb = cb0 + s * _SUP
    pltpu.async_copy(gidx3.at[cid, pl.ds(cb, _SUP)], gib.at[b], sxs[b])
    pltpu.async_copy(sidx3.at[pl.ds(cb, _SUP)], sib.at[b], sxs[b])


def _idx_wait(gidx3, sidx3, gib, sib, sxs, cb0, s, b, cid):
    cb = cb0 + s * _SUP
    pltpu.make_async_copy(gidx3.at[cid, pl.ds(cb, _SUP)], gib.at[b], sxs[b]).wait()
    pltpu.make_async_copy(sidx3.at[pl.ds(cb, _SUP)], sib.at[b], sxs[b]).wait()


def _make_sc_pass(fh, n_chunks, interpret=False):
    @functools.partial(
        pl.kernel,
        out_type=jax.ShapeDtypeStruct((2, _NP, fh), jnp.float32),
        mesh=_sc_mesh(),
        scratch_types=[
            pltpu.VMEM((2, _SUP, _CHUNK), jnp.int32),
            pltpu.VMEM((2, _SUP, _CHUNK), jnp.int32),
            pltpu.VMEM((4, _CHUNK, fh), jnp.float32),
            pltpu.VMEM_SHARED((_NP, fh), jnp.float32),
            pltpu.SemaphoreType.DMA,
            pltpu.SemaphoreType.DMA,
            pltpu.SemaphoreType.DMA,
            pltpu.SemaphoreType.DMA,
            pltpu.SemaphoreType.DMA,
            pltpu.SemaphoreType.DMA,
            pltpu.SemaphoreType.DMA,
            pltpu.SemaphoreType.DMA,
            pltpu.SemaphoreType.DMA,
            pltpu.SemaphoreType.DMA,
        ],
        compiler_params=pltpu.CompilerParams(use_tc_tiling_on_sc=False),
        interpret=interpret,
    )
    def pass_kernel(table, gidx3, sidx3, zrows, out,
                    gib, sib, rowsb, acc,
                    sx0, sx1, sg0, sg1, sg2, sg3, ss0, ss1, ss2, ss3):
        cid = lax.axis_index("c")
        sid = lax.axis_index("s")
        sxs = (sx0, sx1)
        sgs = (sg0, sg1, sg2, sg3)
        sss = (ss0, ss1, ss2, ss3)
        rbase = sid * _RPT
        cb0 = sid * n_chunks
        _idx_start(gidx3, sidx3, gib, sib, sxs, cb0, 0, 0, cid)
        pltpu.sync_copy(zrows.at[pl.ds(rbase, _RPT)], acc.at[pl.ds(rbase, _RPT)])
        plsc.subcore_barrier()
        _edge_pipeline(table, gidx3, sidx3, gib, sib, rowsb, acc,
                       sxs, sgs, sss, n_chunks, cb0, cid)
        _idx_wait(gidx3, sidx3, gib, sib, sxs, cb0, 0, 0, cid)
        plsc.subcore_barrier()
        pltpu.sync_copy(acc.at[pl.ds(rbase, _RPT)], out.at[cid, pl.ds(rbase, _RPT)])

    return pass_kernel


def _make_sc_appnp(interpret=False):
    fh = 32
    nch = _NCH_G
    nblk = _RPT // _CHUNK

    @functools.partial(
        pl.kernel,
        out_type=[jax.ShapeDtypeStruct((2, _NP, fh), jnp.float32),
                  jax.ShapeDtypeStruct((2 * _NP, fh), jnp.float32)],
        mesh=_sc_mesh(),
        scratch_types=[
            pltpu.VMEM((2, _SUP, _CHUNK), jnp.int32),
            pltpu.VMEM((2, _SUP, _CHUNK), jnp.int32),
            pltpu.VMEM((4, _CHUNK, fh), jnp.float32),
            pltpu.VMEM((_RPT, fh), jnp.float32),
            pltpu.VMEM((_RPT, fh), jnp.float32),
            pltpu.VMEM((_RPT, 16), jnp.float32),
            pltpu.VMEM((_CHUNK, fh), jnp.float32),
            pltpu.VMEM((_CHUNK, fh), jnp.float32),
            pltpu.VMEM_SHARED((_NP, fh), jnp.float32),
            pltpu.SemaphoreType.DMA,
            pltpu.SemaphoreType.DMA,
            pltpu.SemaphoreType.DMA,
            pltpu.SemaphoreType.DMA,
            pltpu.SemaphoreType.DMA,
            pltpu.SemaphoreType.DMA,
            pltpu.SemaphoreType.DMA,
            pltpu.SemaphoreType.DMA,
            pltpu.SemaphoreType.DMA,
            pltpu.SemaphoreType.DMA,
        ],
        compiler_params=pltpu.CompilerParams(use_tc_tiling_on_sc=False),
        interpret=interpret,
    )
    def appnp_kernel(th, evs, dv16, gidx3, sidx3, zrows, outT, tscr,
                     gib, sib, rowsb, tbuf, evsbuf, dvbuf, abuf, zbuf, acc,
                     sx0, sx1, sg0, sg1, sg2, sg3, ss0, ss1, ss2, ss3):
        cid = lax.axis_index("c")
        sid = lax.axis_index("s")
        sxs = (sx0, sx1)
        sgs = (sg0, sg1, sg2, sg3)
        sss = (ss0, ss1, ss2, ss3)
        rbase = sid * _RPT
        cb0 = sid * nch
        tb = cid * _NP + rbase
        _idx_start(gidx3, sidx3, gib, sib, sxs, cb0, 0, 0, cid)
        pltpu.sync_copy(zrows.at[pl.ds(rbase, _RPT)], acc.at[pl.ds(rbase, _RPT)])
        pltpu.sync_copy(th.at[cid, pl.ds(rbase, _RPT)], tbuf)
        pltpu.sync_copy(evs.at[cid, pl.ds(rbase, _RPT)], evsbuf)
        pltpu.sync_copy(dv16.at[pl.ds(rbase, _RPT)], dvbuf)
        pltpu.sync_copy(zrows.at[pl.ds(0, _CHUNK)], zbuf)
        pltpu.sync_copy(tbuf, tscr.at[pl.ds(tb, _RPT)])

        def iter_body(k, carry):
            plsc.subcore_barrier()
            _edge_pipeline(tscr, gidx3, sidx3, gib, sib, rowsb, acc,
                           sxs, sgs, sss, nch, cb0, cid)
            plsc.subcore_barrier()
            for blk in range(nblk):
                row0 = rbase + blk * _CHUNK
                pltpu.sync_copy(acc.at[pl.ds(row0, _CHUNK)], abuf)

                def grp_body(g, c2, _blk=blk):
                    for u in range(16):
                        la = g * 16 + u
                        lr = _blk * _CHUNK + la
                        dv = dvbuf[lr, :]
                        for c in range(fh // 16):
                            sl = pl.ds(c * 16, 16)
                            tbuf[lr, sl] = dv * (abuf[la, sl] + tbuf[lr, sl]) + evsbuf[lr, sl]
                    return c2

                lax.fori_loop(0, _CHUNK // 16, grp_body, 0)
                pltpu.sync_copy(zbuf, acc.at[pl.ds(row0, _CHUNK)])
            pltpu.sync_copy(tbuf, tscr.at[pl.ds(tb, _RPT)])
            return carry

        lax.fori_loop(0, _APPNP_K, iter_body, 0)
        _idx_wait(gidx3, sidx3, gib, sib, sxs, cb0, 0, 0, cid)
        pltpu.sync_copy(tbuf, outT.at[cid, pl.ds(rbase, _RPT)])

    return appnp_kernel


def _make_sc_degree(interpret=False):
    nsup = _NCH_D // _SUP

    @functools.partial(
        pl.kernel,
        out_type=jax.ShapeDtypeStruct((2, _NP, 16), jnp.float32),
        mesh=_sc_mesh(),
        scratch_types=[
            pltpu.VMEM((_SUP, _CHUNK), jnp.int32),
            pltpu.VMEM((_SUP, _CHUNK), jnp.int32),
            pltpu.VMEM((_CHUNK, 16), jnp.float32),
            pltpu.VMEM_SHARED((_NP, 16), jnp.float32),
            pltpu.SemaphoreType.DMA,
            pltpu.SemaphoreType.DMA,
            pltpu.SemaphoreType.DMA,
        ],
        compiler_params=pltpu.CompilerParams(use_tc_tiling_on_sc=False),
        interpret=interpret,
    )
    def deg_kernel(didx3, ones_hbm, zrows, out, si0, si1, ones_v, acc, sx0, sx1, ss):
        cid = lax.axis_index("c")
        sid = lax.axis_index("s")
        sis = (si0, si1)
        sxs = (sx0, sx1)
        rbase = sid * _RPT
        pltpu.sync_copy(zrows.at[pl.ds(rbase, _RPT)], acc.at[pl.ds(rbase, _RPT)])
        pltpu.sync_copy(ones_hbm, ones_v)
        plsc.subcore_barrier()
        wid = cid * 16 + sid
        cb0 = wid * _NCH_D

        def idx_start(s, b):
            pltpu.async_copy(didx3.at[pl.ds(cb0 + s * _SUP, _SUP)], sis[b], sxs[b])

        def idx_wait(s, b):
            pltpu.make_async_copy(
                didx3.at[pl.ds(cb0 + s * _SUP, _SUP)], sis[b], sxs[b]).wait()

        idx_start(0, 0)

        def sup_body(t, carry):
            for ph in range(2):
                s = 2 * t + ph
                idx_wait(s, ph)
                s_next = jnp.where(s + 1 == nsup, 0, s + 1)
                idx_start(s_next, 1 - ph)
                descs = [
                    pltpu.async_copy(ones_v, acc.at[sis[ph].at[j]], ss, add=True)
                    for j in range(_SUP)
                ]
                for d in descs:
                    d.wait()
            return carry

        lax.fori_loop(0, nsup // 2, sup_body, 0)
        idx_wait(0, 0)
        plsc.subcore_barrier()
        pltpu.sync_copy(acc.at[pl.ds(rbase, _RPT)], out.at[cid, pl.ds(rbase, _RPT)])

    return deg_kernel


_R = 1000


def _tc_call(body, out_shapes, in_specs, out_specs, interpret=False):
    return pl.pallas_call(
        body,
        out_shape=out_shapes,
        grid=(_N // _R,),
        in_specs=in_specs,
        out_specs=out_specs,
        interpret=interpret,
    )


def _spec_rows(width):
    return pl.BlockSpec((_R, width), lambda i: (i, 0))


def _spec_halves(width):
    return pl.BlockSpec((2, _R, width), lambda i: (0, i, 0))


def _spec_full(a, b):
    return pl.BlockSpec((a, b), lambda i: (0, 0))


def _tc1(x, w1, degp, interpret=False):
    def body(x_ref, w_ref, d_ref, dinv_ref, t_ref):
        deg = d_ref[0, :, 0] + d_ref[1, :, 0] + 1.0
        dinv = lax.rsqrt(deg)[:, None]
        dinv_ref[...] = dinv
        y = jnp.dot(x_ref[...], w_ref[...], preferred_element_type=jnp.float32)
        t = y * dinv
        t_ref[0] = t[:, :64]
        t_ref[1] = t[:, 64:]

    return _tc_call(
        body,
        [jax.ShapeDtypeStruct((_N, 1), jnp.float32),
         jax.ShapeDtypeStruct((2, _N, 64), jnp.float32)],
        [_spec_rows(_H), _spec_full(_H, _H), _spec_halves(16)],
        [pl.BlockSpec((_R, 1), lambda i: (i, 0)), _spec_halves(64)],
        interpret,
    )(x, w1, degp)


def _tc2(agg, t0h, b1, dinv, interpret=False):
    def body(a_ref, t_ref, b_ref, dinv_ref, h_ref):
        d = dinv_ref[...]
        for c in range(2):
            v = d * (a_ref[c] + t_ref[c]) + b_ref[0, c * 64:(c + 1) * 64]
            h_ref[c] = jnp.maximum(v, 0.0)

    return _tc_call(
        body,
        jax.ShapeDtypeStruct((2, _N, 64), jnp.float32),
        [_spec_halves(64), _spec_halves(64), _spec_full(1, _H),
         pl.BlockSpec((_R, 1), lambda i: (i, 0))],
        _spec_halves(64),
        interpret,
    )(agg, t0h, b1, dinv)


def _tc3(crf, h1h, w2, dinv, interpret=False):
    def body(c_ref, h_ref, w_ref, dinv_ref, t_ref):
        h1 = jnp.concatenate([h_ref[0], h_ref[1]], axis=1)
        cr = jnp.concatenate([c_ref[0], c_ref[1]], axis=1)
        hc = (1.0 - _CRF_ALPHA) * h1 + _CRF_ALPHA * cr
        t = dinv_ref[...] * jnp.dot(hc, w_ref[...], preferred_element_type=jnp.float32)
        t_ref[0] = t[:, :64]
        t_ref[1] = t[:, 64:]

    return _tc_call(
        body,
        jax.ShapeDtypeStruct((2, _N, 64), jnp.float32),
        [_spec_halves(64), _spec_halves(64), _spec_full(_H, _H),
         pl.BlockSpec((_R, 1), lambda i: (i, 0))],
        _spec_halves(64),
        interpret,
    )(crf, h1h, w2, dinv)


def _tc4(agg, t1h, b2, wc, bc, we, be, dinv, interpret=False):
    def body(a_ref, t_ref, b2_ref, wc_ref, bc_ref, we_ref, be_ref, dinv_ref,
             th_ref, evs_ref, dv_ref):
        d = dinv_ref[...]
        hs = []
        for c in range(2):
            v = d * (a_ref[c] + t_ref[c]) + b2_ref[0, c * 64:(c + 1) * 64]
            hs.append(jnp.maximum(v, 0.0))
        h2 = jnp.concatenate(hs, axis=1)
        logits = jnp.dot(h2, wc_ref[...], preferred_element_type=jnp.float32) + bc_ref[0]
        ev = jnp.maximum(
            jnp.dot(logits, we_ref[...], preferred_element_type=jnp.float32) + be_ref[0],
            0.0)
        t = d * ev
        th_ref[0] = t[:, :32]
        th_ref[1] = t[:, 32:]
        evs = _APPNP_ALPHA * t
        evs_ref[0] = evs[:, :32]
        evs_ref[1] = evs[:, 32:]
        dv_ref[...] = jnp.broadcast_to((1.0 - _APPNP_ALPHA) * d * d, (_R, 16))

    return _tc_call(
        body,
        [jax.ShapeDtypeStruct((2, _NP, 32), jnp.float32),
         jax.ShapeDtypeStruct((2, _NP, 32), jnp.float32),
         jax.ShapeDtypeStruct((_NP, 16), jnp.float32)],
        [_spec_halves(64), _spec_halves(64), _spec_full(1, _H),
         _spec_full(_H, _C), _spec_full(1, _C), _spec_full(_C, _C),
         _spec_full(1, _C), pl.BlockSpec((_R, 1), lambda i: (i, 0))],
        [_spec_halves(32), _spec_halves(32), pl.BlockSpec((_R, 16), lambda i: (i, 0))],
        interpret,
    )(agg, t1h, b2, wc, bc, we, be, dinv)


def _tc6(tk, dinv, interpret=False):
    def body(t_ref, dinv_ref, o_ref):
        d = dinv_ref[...]
        h = jnp.concatenate([t_ref[0], t_ref[1]], axis=1) / d
        m = jnp.max(h, axis=1, keepdims=True)
        z = h - m
        lse = jnp.log(jnp.sum(jnp.exp(z), axis=1, keepdims=True))
        o_ref[...] = z - lse

    return _tc_call(
        body,
        jax.ShapeDtypeStruct((_N, _C), jnp.float32),
        [_spec_halves(32), pl.BlockSpec((_R, 1), lambda i: (i, 0))],
        _spec_rows(_C),
        interpret,
    )(tk, dinv)


def _run(x, edge_index, W1, b1, W2, b2, Wc, bc, We, be, interpret=False):
    src = edge_index[0]
    dst = edge_index[1]
    i32 = jnp.int32

    pad_g = _EP_G - _E
    gidx_g = jnp.concatenate([src, jnp.zeros((pad_g,), i32)])
    sidx_g = jnp.concatenate([dst, jnp.full((pad_g,), _N, i32)])
    pad_c = _EP_C - 2 * _E
    gidx_c = jnp.concatenate([dst, src, jnp.zeros((pad_c,), i32)])
    sidx_c = jnp.concatenate([src, dst, jnp.full((pad_c,), _N, i32)])
    pad_d = _EP_D - _E
    didx = jnp.concatenate([dst, jnp.full((pad_d,), _N, i32)])

    gidx_a = jnp.stack([gidx_g, gidx_g + _NP]).reshape(2, _EP_G // _CHUNK, _CHUNK)
    gidx_g = jnp.stack([gidx_g, gidx_g + _N]).reshape(2, _EP_G // _CHUNK, _CHUNK)
    sidx_g = sidx_g.reshape(_EP_G // _CHUNK, _CHUNK)
    gidx_c = jnp.stack([gidx_c, gidx_c + _N]).reshape(2, _EP_C // _CHUNK, _CHUNK)
    sidx_c = sidx_c.reshape(_EP_C // _CHUNK, _CHUNK)
    didx = didx.reshape(_EP_D // _CHUNK, _CHUNK)

    z64 = jnp.zeros((_NP, 64), jnp.float32)
    z32 = jnp.zeros((_NP, 32), jnp.float32)
    z16 = jnp.zeros((_NP, 16), jnp.float32)
    on16 = jnp.ones((_CHUNK, 16), jnp.float32)

    b1r = b1.reshape(1, _H)
    b2r = b2.reshape(1, _H)
    bcr = bc.reshape(1, _C)
    ber = be.reshape(1, _C)

    pass64 = _make_sc_pass(64, _NCH_G, interpret)
    pass64c = _make_sc_pass(64, _NCH_C, interpret)

    degp = _make_sc_degree(interpret)(didx, on16, z16)
    dinv, t0h = _tc1(x, W1, degp, interpret)

    agg1 = pass64(t0h.reshape(2 * _N, 64), gidx_g, sidx_g, z64)
    h1h = _tc2(agg1, t0h, b1r, dinv, interpret)

    crf = pass64c(h1h.reshape(2 * _N, 64), gidx_c, sidx_c, z64)
    t1h = _tc3(crf, h1h, W2, dinv, interpret)

    agg2 = pass64(t1h.reshape(2 * _N, 64), gidx_g, sidx_g, z64)
    th, evs, dv16 = _tc4(agg2, t1h, b2r, Wc, bcr, We, ber, dinv, interpret)

    tk, _ = _make_sc_appnp(interpret)(th, evs, dv16, gidx_a, sidx_g, z32)
    return _tc6(tk, dinv, interpret)


def kernel(x, edge_index, W1, b1, W2, b2, Wc, bc, We, be):
    return _run(x, edge_index, W1, b1, W2, b2, Wc, bc, We, be)


# speedup vs baseline: 15.5246x; 1.5008x over previous
import functools

import jax
import jax.numpy as jnp
from jax import lax
from jax.experimental import pallas as pl
from jax.experimental.pallas import tpu as pltpu
from jax.experimental.pallas import tpu_sc as plsc

_N = 10000
_NP = 10240
_E = 320000
_H = 128
_C = 64
_CRF_ALPHA = 0.1
_APPNP_K = 10
_APPNP_ALPHA = 0.1

_CHUNK = 128
_RPT = _NP // 16

_SUP = 40


def _round_chunks(n_edges, n_workers):
    per = -(-n_edges // (n_workers * _CHUNK))
    per = -(-per // (2 * _SUP)) * (2 * _SUP)
    return per


_NCH_G = _round_chunks(_E, 16)
_EP_G = _NCH_G * 16 * _CHUNK
_NCH_C = _round_chunks(2 * _E, 16)
_EP_C = _NCH_C * 16 * _CHUNK
_NCH_D = _round_chunks(_E, 32)
_EP_D = _NCH_D * 32 * _CHUNK


def _sc_mesh():
    return plsc.VectorSubcoreMesh(
        core_axis_name="c", subcore_axis_name="s", num_cores=2, num_subcores=16)


def _edge_pipeline(table, gidx3, sidx3, gib, sib, rowsb, acc,
                   sxs, sgs, sss, n_chunks, cb0, cid, rb16=None):
    nsup = n_chunks // _SUP
    gbuf = rowsb if rb16 is None else rb16
    fh = rowsb.shape[2]

    def sup_body(t, carry):
        for ph in range(2):
            s = 2 * t + ph
            _idx_wait(gidx3, sidx3, gib, sib, sxs, cb0, s, ph, cid)
            s_next = jnp.where(s + 1 == nsup, 0, s + 1)
            _idx_start(gidx3, sidx3, gib, sib, sxs, cb0, s_next, 1 - ph, cid)
            gd = [None] * _SUP
            sd = [None] * _SUP

            def scat(j):
                gd[j].wait()
                b = j % 4
                if rb16 is not None:
                    def unp(g, carry2):
                        for u in range(16):
                            r = g * 16 + u
                            for h in range(fh // 32):
                                v = rb16[b, r, pl.ds(h * 32, 32)]
                                a1, a2 = plsc.unpack(
                                    v, format=plsc.PackFormat.INTERLEAVED)
                                rowsb[b, r, pl.ds(h * 32, 16)] = a1
                                rowsb[b, r, pl.ds(h * 32 + 16, 16)] = a2
                        return carry2
                    lax.fori_loop(0, _CHUNK // 16, unp, 0)
                sd[j] = pltpu.async_copy(
                    rowsb.at[b], acc.at[sib.at[ph, j]], sss[b], add=True)

            for j in range(_SUP):
                b = j % 4
                if j >= 4:
                    sd[j - 4].wait()
                gd[j] = pltpu.async_copy(
                    table.at[gib.at[ph, j]], gbuf.at[b], sgs[b])
                if j >= 2:
                    scat(j - 2)
            scat(_SUP - 2)
            scat(_SUP - 1)
            for j in range(_SUP - 4, _SUP):
                sd[j].wait()
        return carry

    lax.fori_loop(0, nsup // 2, sup_body, 0)


def _idx_start(gidx3, sidx3, gib, sib, sxs, cb0, s, b, cid):
    cb = cb0 + s * _SUP
    pltpu.async_copy(gidx3.at[cid, pl.ds(cb, _SUP)], gib.at[b], sxs[b])
    pltpu.async_copy(sidx3.at[pl.ds(cb, _SUP)], sib.at[b], sxs[b])


def _idx_wait(gidx3, sidx3, gib, sib, sxs, cb0, s, b, cid):
    cb = cb0 + s * _SUP
    pltpu.make_async_copy(gidx3.at[cid, pl.ds(cb, _SUP)], gib.at[b], sxs[b]).wait()
    pltpu.make_async_copy(sidx3.at[pl.ds(cb, _SUP)], sib.at[b], sxs[b]).wait()


def _make_sc_pass(fh, n_chunks, interpret=False):
    @functools.partial(
        pl.kernel,
        out_type=jax.ShapeDtypeStruct((2, _NP, fh), jnp.float32),
        mesh=_sc_mesh(),
        scratch_types=[
            pltpu.VMEM((2, _SUP, _CHUNK), jnp.int32),
            pltpu.VMEM((2, _SUP, _CHUNK), jnp.int32),
            pltpu.VMEM((4, _CHUNK, fh), jnp.float32),
            pltpu.VMEM_SHARED((_NP, fh), jnp.float32),
            pltpu.SemaphoreType.DMA,
            pltpu.SemaphoreType.DMA,
            pltpu.SemaphoreType.DMA,
            pltpu.SemaphoreType.DMA,
            pltpu.SemaphoreType.DMA,
            pltpu.SemaphoreType.DMA,
            pltpu.SemaphoreType.DMA,
            pltpu.SemaphoreType.DMA,
            pltpu.SemaphoreType.DMA,
            pltpu.SemaphoreType.DMA,
        ],
        compiler_params=pltpu.CompilerParams(use_tc_tiling_on_sc=False),
        interpret=interpret,
    )
    def pass_kernel(table, gidx3, sidx3, zrows, out,
                    gib, sib, rowsb, acc,
                    sx0, sx1, sg0, sg1, sg2, sg3, ss0, ss1, ss2, ss3):
        cid = lax.axis_index("c")
        sid = lax.axis_index("s")
        sxs = (sx0, sx1)
        sgs = (sg0, sg1, sg2, sg3)
        sss = (ss0, ss1, ss2, ss3)
        rbase = sid * _RPT
        cb0 = sid * n_chunks
        _idx_start(gidx3, sidx3, gib, sib, sxs, cb0, 0, 0, cid)
        pltpu.sync_copy(zrows.at[pl.ds(rbase, _RPT)], acc.at[pl.ds(rbase, _RPT)])
        plsc.subcore_barrier()
        _edge_pipeline(table, gidx3, sidx3, gib, sib, rowsb, acc,
                       sxs, sgs, sss, n_chunks, cb0, cid)
        _idx_wait(gidx3, sidx3, gib, sib, sxs, cb0, 0, 0, cid)
        plsc.subcore_barrier()
        pltpu.sync_copy(acc.at[pl.ds(rbase, _RPT)], out.at[cid, pl.ds(rbase, _RPT)])

    return pass_kernel


def _make_sc_appnp(interpret=False):
    fh = 32
    nch = _NCH_G
    nblk = _RPT // _CHUNK

    @functools.partial(
        pl.kernel,
        out_type=jax.ShapeDtypeStruct((2, _NP, fh), jnp.float32),
        mesh=_sc_mesh(),
        scratch_types=[
            pltpu.VMEM((2, _SUP, _CHUNK), jnp.int32),
            pltpu.VMEM((2, _SUP, _CHUNK), jnp.int32),
            pltpu.VMEM((4, _CHUNK, fh), jnp.float32),
            pltpu.VMEM((_RPT, fh), jnp.float32),
            pltpu.VMEM((_CHUNK, fh), jnp.float32),
            pltpu.VMEM((_CHUNK, 16), jnp.float32),
            pltpu.VMEM((_CHUNK, fh), jnp.float32),
            pltpu.VMEM((_CHUNK, fh), jnp.float32),
            pltpu.VMEM_SHARED((_NP, fh), jnp.float32),
            pltpu.VMEM_SHARED((_NP, fh), jnp.float32),
            pltpu.SemaphoreType.DMA,
            pltpu.SemaphoreType.DMA,
            pltpu.SemaphoreType.DMA,
            pltpu.SemaphoreType.DMA,
            pltpu.SemaphoreType.DMA,
            pltpu.SemaphoreType.DMA,
            pltpu.SemaphoreType.DMA,
            pltpu.SemaphoreType.DMA,
            pltpu.SemaphoreType.DMA,
            pltpu.SemaphoreType.DMA,
        ],
        compiler_params=pltpu.CompilerParams(use_tc_tiling_on_sc=False),
        interpret=interpret,
    )
    def appnp_kernel(th, evs, dv16, gidx3, sidx3, zrows, outT,
                     gib, sib, rowsb, tbuf, evb, dvb, abuf, zbuf, acc, tsh,
                     sx0, sx1, sg0, sg1, sg2, sg3, ss0, ss1, ss2, ss3):
        cid = lax.axis_index("c")
        sid = lax.axis_index("s")
        sxs = (sx0, sx1)
        sgs = (sg0, sg1, sg2, sg3)
        sss = (ss0, ss1, ss2, ss3)
        rbase = sid * _RPT
        cb0 = sid * nch
        _idx_start(gidx3, sidx3, gib, sib, sxs, cb0, 0, 0, cid)
        pltpu.sync_copy(zrows.at[pl.ds(rbase, _RPT)], acc.at[pl.ds(rbase, _RPT)])
        pltpu.sync_copy(th.at[cid, pl.ds(rbase, _RPT)], tbuf)
        pltpu.sync_copy(zrows.at[pl.ds(0, _CHUNK)], zbuf)
        pltpu.sync_copy(tbuf, tsh.at[pl.ds(rbase, _RPT)])

        def iter_body(k, carry):
            plsc.subcore_barrier()
            _edge_pipeline(tsh, gidx3, sidx3, gib, sib, rowsb, acc,
                           sxs, sgs, sss, nch, cb0, cid)
            plsc.subcore_barrier()
            for blk in range(nblk):
                row0 = rbase + blk * _CHUNK
                pltpu.sync_copy(acc.at[pl.ds(row0, _CHUNK)], abuf)
                pltpu.sync_copy(evs.at[cid, pl.ds(row0, _CHUNK)], evb)
                pltpu.sync_copy(dv16.at[pl.ds(row0, _CHUNK)], dvb)

                def grp_body(g, c2, _blk=blk):
                    for u in range(16):
                        la = g * 16 + u
                        lr = _blk * _CHUNK + la
                        dv = dvb[la, :]
                        for c in range(fh // 16):
                            sl = pl.ds(c * 16, 16)
                            tbuf[lr, sl] = dv * (abuf[la, sl] + tbuf[lr, sl]) + evb[la, sl]
                    return c2

                lax.fori_loop(0, _CHUNK // 16, grp_body, 0)
                pltpu.sync_copy(zbuf, acc.at[pl.ds(row0, _CHUNK)])
            pltpu.sync_copy(tbuf, tsh.at[pl.ds(rbase, _RPT)])
            return carry

        lax.fori_loop(0, _APPNP_K, iter_body, 0)
        _idx_wait(gidx3, sidx3, gib, sib, sxs, cb0, 0, 0, cid)
        pltpu.sync_copy(tbuf, outT.at[cid, pl.ds(rbase, _RPT)])

    return appnp_kernel


def _make_sc_degree(interpret=False):
    nsup = _NCH_D // _SUP

    @functools.partial(
        pl.kernel,
        out_type=jax.ShapeDtypeStruct((2, _NP, 16), jnp.float32),
        mesh=_sc_mesh(),
        scratch_types=[
            pltpu.VMEM((_SUP, _CHUNK), jnp.int32),
            pltpu.VMEM((_SUP, _CHUNK), jnp.int32),
            pltpu.VMEM((_CHUNK, 16), jnp.float32),
            pltpu.VMEM_SHARED((_NP, 16), jnp.float32),
            pltpu.SemaphoreType.DMA,
            pltpu.SemaphoreType.DMA,
            pltpu.SemaphoreType.DMA,
        ],
        compiler_params=pltpu.CompilerParams(use_tc_tiling_on_sc=False),
        interpret=interpret,
    )
    def deg_kernel(didx3, ones_hbm, zrows, out, si0, si1, ones_v, acc, sx0, sx1, ss):
        cid = lax.axis_index("c")
        sid = lax.axis_index("s")
        sis = (si0, si1)
        sxs = (sx0, sx1)
        rbase = sid * _RPT
        pltpu.sync_copy(zrows.at[pl.ds(rbase, _RPT)], acc.at[pl.ds(rbase, _RPT)])
        pltpu.sync_copy(ones_hbm, ones_v)
        plsc.subcore_barrier()
        wid = cid * 16 + sid
        cb0 = wid * _NCH_D

        def idx_start(s, b):
            pltpu.async_copy(didx3.at[pl.ds(cb0 + s * _SUP, _SUP)], sis[b], sxs[b])

        def idx_wait(s, b):
            pltpu.make_async_copy(
                didx3.at[pl.ds(cb0 + s * _SUP, _SUP)], sis[b], sxs[b]).wait()

        idx_start(0, 0)

        def sup_body(t, carry):
            for ph in range(2):
                s = 2 * t + ph
                idx_wait(s, ph)
                s_next = jnp.where(s + 1 == nsup, 0, s + 1)
                idx_start(s_next, 1 - ph)
                descs = [
                    pltpu.async_copy(ones_v, acc.at[sis[ph].at[j]], ss, add=True)
                    for j in range(_SUP)
                ]
                for d in descs:
                    d.wait()
            return carry

        lax.fori_loop(0, nsup // 2, sup_body, 0)
        idx_wait(0, 0)
        plsc.subcore_barrier()
        pltpu.sync_copy(acc.at[pl.ds(rbase, _RPT)], out.at[cid, pl.ds(rbase, _RPT)])

    return deg_kernel


_R = 1000


def _tc_call(body, out_shapes, in_specs, out_specs, interpret=False):
    return pl.pallas_call(
        body,
        out_shape=out_shapes,
        grid=(_N // _R,),
        in_specs=in_specs,
        out_specs=out_specs,
        interpret=interpret,
    )


def _spec_rows(width):
    return pl.BlockSpec((_R, width), lambda i: (i, 0))


def _spec_halves(width):
    return pl.BlockSpec((2, _R, width), lambda i: (0, i, 0))


def _spec_full(a, b):
    return pl.BlockSpec((a, b), lambda i: (0, 0))


def _tc1(x, w1, degp, interpret=False):
    def body(x_ref, w_ref, d_ref, dinv_ref, t_ref):
        deg = d_ref[0, :, 0] + d_ref[1, :, 0] + 1.0
        dinv = lax.rsqrt(deg)[:, None]
        dinv_ref[...] = dinv
        y = jnp.dot(x_ref[...], w_ref[...], preferred_element_type=jnp.float32)
        t = y * dinv
        t_ref[0] = t[:, :64]
        t_ref[1] = t[:, 64:]

    return _tc_call(
        body,
        [jax.ShapeDtypeStruct((_N, 1), jnp.float32),
         jax.ShapeDtypeStruct((2, _N, 64), jnp.float32)],
        [_spec_rows(_H), _spec_full(_H, _H), _spec_halves(16)],
        [pl.BlockSpec((_R, 1), lambda i: (i, 0)), _spec_halves(64)],
        interpret,
    )(x, w1, degp)


def _tc2(agg, t0h, b1, dinv, interpret=False):
    def body(a_ref, t_ref, b_ref, dinv_ref, h_ref):
        d = dinv_ref[...]
        for c in range(2):
            v = d * (a_ref[c] + t_ref[c]) + b_ref[0, c * 64:(c + 1) * 64]
            h_ref[c] = jnp.maximum(v, 0.0)

    return _tc_call(
        body,
        jax.ShapeDtypeStruct((2, _N, 64), jnp.float32),
        [_spec_halves(64), _spec_halves(64), _spec_full(1, _H),
         pl.BlockSpec((_R, 1), lambda i: (i, 0))],
        _spec_halves(64),
        interpret,
    )(agg, t0h, b1, dinv)


def _tc3(crf, h1h, w2, dinv, interpret=False):
    def body(c_ref, h_ref, w_ref, dinv_ref, t_ref):
        h1 = jnp.concatenate([h_ref[0], h_ref[1]], axis=1)
        cr = jnp.concatenate([c_ref[0], c_ref[1]], axis=1)
        hc = (1.0 - _CRF_ALPHA) * h1 + _CRF_ALPHA * cr
        t = dinv_ref[...] * jnp.dot(hc, w_ref[...], preferred_element_type=jnp.float32)
        t_ref[0] = t[:, :64]
        t_ref[1] = t[:, 64:]

    return _tc_call(
        body,
        jax.ShapeDtypeStruct((2, _N, 64), jnp.float32),
        [_spec_halves(64), _spec_halves(64), _spec_full(_H, _H),
         pl.BlockSpec((_R, 1), lambda i: (i, 0))],
        _spec_halves(64),
        interpret,
    )(crf, h1h, w2, dinv)


def _tc4(agg, t1h, b2, wc, bc, we, be, dinv, interpret=False):
    def body(a_ref, t_ref, b2_ref, wc_ref, bc_ref, we_ref, be_ref, dinv_ref,
             th_ref, evs_ref, dv_ref):
        d = dinv_ref[...]
        hs = []
        for c in range(2):
            v = d * (a_ref[c] + t_ref[c]) + b2_ref[0, c * 64:(c + 1) * 64]
            hs.append(jnp.maximum(v, 0.0))
        h2 = jnp.concatenate(hs, axis=1)
        logits = jnp.dot(h2, wc_ref[...], preferred_element_type=jnp.float32) + bc_ref[0]
        ev = jnp.maximum(
            jnp.dot(logits, we_ref[...], preferred_element_type=jnp.float32) + be_ref[0],
            0.0)
        t = d * ev
        th_ref[0] = t[:, :32]
        th_ref[1] = t[:, 32:]
        evs = _APPNP_ALPHA * t
        evs_ref[0] = evs[:, :32]
        evs_ref[1] = evs[:, 32:]
        dv_ref[...] = jnp.broadcast_to((1.0 - _APPNP_ALPHA) * d * d, (_R, 16))

    return _tc_call(
        body,
        [jax.ShapeDtypeStruct((2, _NP, 32), jnp.float32),
         jax.ShapeDtypeStruct((2, _NP, 32), jnp.float32),
         jax.ShapeDtypeStruct((_NP, 16), jnp.float32)],
        [_spec_halves(64), _spec_halves(64), _spec_full(1, _H),
         _spec_full(_H, _C), _spec_full(1, _C), _spec_full(_C, _C),
         _spec_full(1, _C), pl.BlockSpec((_R, 1), lambda i: (i, 0))],
        [_spec_halves(32), _spec_halves(32), pl.BlockSpec((_R, 16), lambda i: (i, 0))],
        interpret,
    )(agg, t1h, b2, wc, bc, we, be, dinv)


def _tc6(tk, dinv, interpret=False):
    def body(t_ref, dinv_ref, o_ref):
        d = dinv_ref[...]
        h = jnp.concatenate([t_ref[0], t_ref[1]], axis=1) / d
        m = jnp.max(h, axis=1, keepdims=True)
        z = h - m
        lse = jnp.log(jnp.sum(jnp.exp(z), axis=1, keepdims=True))
        o_ref[...] = z - lse

    return _tc_call(
        body,
        jax.ShapeDtypeStruct((_N, _C), jnp.float32),
        [_spec_halves(32), pl.BlockSpec((_R, 1), lambda i: (i, 0))],
        _spec_rows(_C),
        interpret,
    )(tk, dinv)


def _run(x, edge_index, W1, b1, W2, b2, Wc, bc, We, be, interpret=False):
    src = edge_index[0]
    dst = edge_index[1]
    i32 = jnp.int32

    pad_g = _EP_G - _E
    gidx_g = jnp.concatenate([src, jnp.zeros((pad_g,), i32)])
    sidx_g = jnp.concatenate([dst, jnp.full((pad_g,), _N, i32)])
    pad_c = _EP_C - 2 * _E
    gidx_c = jnp.concatenate([dst, src, jnp.zeros((pad_c,), i32)])
    sidx_c = jnp.concatenate([src, dst, jnp.full((pad_c,), _N, i32)])
    pad_d = _EP_D - _E
    didx = jnp.concatenate([dst, jnp.full((pad_d,), _N, i32)])

    gidx_a = jnp.stack([gidx_g, gidx_g]).reshape(2, _EP_G // _CHUNK, _CHUNK)
    gidx_g = jnp.stack([gidx_g, gidx_g + _N]).reshape(2, _EP_G // _CHUNK, _CHUNK)
    sidx_g = sidx_g.reshape(_EP_G // _CHUNK, _CHUNK)
    gidx_c = jnp.stack([gidx_c, gidx_c + _N]).reshape(2, _EP_C // _CHUNK, _CHUNK)
    sidx_c = sidx_c.reshape(_EP_C // _CHUNK, _CHUNK)
    didx = didx.reshape(_EP_D // _CHUNK, _CHUNK)

    z64 = jnp.zeros((_NP, 64), jnp.float32)
    z32 = jnp.zeros((_NP, 32), jnp.float32)
    z16 = jnp.zeros((_NP, 16), jnp.float32)
    on16 = jnp.ones((_CHUNK, 16), jnp.float32)

    b1r = b1.reshape(1, _H)
    b2r = b2.reshape(1, _H)
    bcr = bc.reshape(1, _C)
    ber = be.reshape(1, _C)

    pass64 = _make_sc_pass(64, _NCH_G, interpret)
    pass64c = _make_sc_pass(64, _NCH_C, interpret)

    degp = _make_sc_degree(interpret)(didx, on16, z16)
    dinv, t0h = _tc1(x, W1, degp, interpret)

    agg1 = pass64(t0h.reshape(2 * _N, 64), gidx_g, sidx_g, z64)
    h1h = _tc2(agg1, t0h, b1r, dinv, interpret)

    crf = pass64c(h1h.reshape(2 * _N, 64), gidx_c, sidx_c, z64)
    t1h = _tc3(crf, h1h, W2, dinv, interpret)

    agg2 = pass64(t1h.reshape(2 * _N, 64), gidx_g, sidx_g, z64)
    th, evs, dv16 = _tc4(agg2, t1h, b2r, Wc, bcr, We, ber, dinv, interpret)

    tk = _make_sc_appnp(interpret)(th, evs, dv16, gidx_a, sidx_g, z32)
    return _tc6(tk, dinv, interpret)


def kernel(x, edge_index, W1, b1, W2, b2, Wc, bc, We, be):
    return _run(x, edge_index, W1, b1, W2, b2, Wc, bc, We, be)


# speedup vs baseline: 21.8278x; 1.4060x over previous
import functools

import jax
import jax.numpy as jnp
from jax import lax
from jax.experimental import pallas as pl
from jax.experimental.pallas import tpu as pltpu
from jax.experimental.pallas import tpu_sc as plsc

_N = 10000
_NP = 10240
_E = 320000
_H = 128
_C = 64
_CRF_ALPHA = 0.1
_APPNP_K = 10
_APPNP_ALPHA = 0.1

_CHUNK = 128
_RPT = _NP // 16

_SUP = 40


def _round_chunks(n_edges, n_workers):
    per = -(-n_edges // (n_workers * _CHUNK))
    per = -(-per // (2 * _SUP)) * (2 * _SUP)
    return per


_NCH_G = _round_chunks(_E, 16)
_EP_G = _NCH_G * 16 * _CHUNK
_NCH_C = _round_chunks(2 * _E, 16)
_EP_C = _NCH_C * 16 * _CHUNK
_NCH_D = _round_chunks(_E, 32)
_EP_D = _NCH_D * 32 * _CHUNK


def _sc_mesh():
    return plsc.VectorSubcoreMesh(
        core_axis_name="c", subcore_axis_name="s", num_cores=2, num_subcores=16)


def _edge_pipeline(table, gidx3, sidx3, gib, sib, rowsb, acc,
                   sxs, sgs, sss, n_chunks, cb0, cid):
    nsup = n_chunks // _SUP
    depth = rowsb.shape[0]
    look = 2 if depth >= 3 else 1

    def sup_body(t, carry):
        for ph in range(2):
            s = 2 * t + ph
            _idx_wait(gidx3, sidx3, gib, sib, sxs, cb0, s, ph, cid)
            s_next = jnp.where(s + 1 == nsup, 0, s + 1)
            _idx_start(gidx3, sidx3, gib, sib, sxs, cb0, s_next, 1 - ph, cid)
            gd = [None] * _SUP
            sd = [None] * _SUP

            def scat(j):
                gd[j].wait()
                b = j % depth
                sd[j] = pltpu.async_copy(
                    rowsb.at[b], acc.at[sib.at[ph, j]], sss[b], add=True)

            for j in range(_SUP):
                b = j % depth
                if j >= depth:
                    sd[j - depth].wait()
                gd[j] = pltpu.async_copy(
                    table.at[gib.at[ph, j]], rowsb.at[b], sgs[b])
                if j >= look:
                    scat(j - look)
            for r in range(look):
                scat(_SUP - look + r)
            for j in range(_SUP - depth, _SUP):
                sd[j].wait()
        return carry

    lax.fori_loop(0, nsup // 2, sup_body, 0)


def _idx_start(gidx3, sidx3, gib, sib, sxs, cb0, s, b, cid):
    cb = cb0 + s * _SUP
    pltpu.async_copy(gidx3.at[cid, pl.ds(cb, _SUP)], gib.at[b], sxs[b])
    pltpu.async_copy(sidx3.at[pl.ds(cb, _SUP)], sib.at[b], sxs[b])


def _idx_wait(gidx3, sidx3, gib, sib, sxs, cb0, s, b, cid):
    cb = cb0 + s * _SUP
    pltpu.make_async_copy(gidx3.at[cid, pl.ds(cb, _SUP)], gib.at[b], sxs[b]).wait()
    pltpu.make_async_copy(sidx3.at[pl.ds(cb, _SUP)], sib.at[b], sxs[b]).wait()


def _make_sc_pass(fh, n_chunks, interpret=False):
    @functools.partial(
        pl.kernel,
        out_type=jax.ShapeDtypeStruct((2, _NP, fh), jnp.float32),
        mesh=_sc_mesh(),
        scratch_types=[
            pltpu.VMEM((2, _SUP, _CHUNK), jnp.int32),
            pltpu.VMEM((2, _SUP, _CHUNK), jnp.int32),
            pltpu.VMEM((2, _CHUNK, fh), jnp.float32),
            pltpu.VMEM_SHARED((_NP, fh), jnp.float32),
            pltpu.VMEM_SHARED((_NP, fh), jnp.float32),
            pltpu.SemaphoreType.DMA,
            pltpu.SemaphoreType.DMA,
            pltpu.SemaphoreType.DMA,
            pltpu.SemaphoreType.DMA,
            pltpu.SemaphoreType.DMA,
            pltpu.SemaphoreType.DMA,
        ],
        compiler_params=pltpu.CompilerParams(use_tc_tiling_on_sc=False),
        interpret=interpret,
    )
    def pass_kernel(table, gidx3, sidx3, zrows, out,
                    gib, sib, rowsb, acc, tsh,
                    sx0, sx1, sg0, sg1, ss0, ss1):
        cid = lax.axis_index("c")
        sid = lax.axis_index("s")
        sxs = (sx0, sx1)
        sgs = (sg0, sg1)
        sss = (ss0, ss1)
        rbase = sid * _RPT
        cb0 = sid * n_chunks
        _idx_start(gidx3, sidx3, gib, sib, sxs, cb0, 0, 0, cid)
        pltpu.sync_copy(zrows.at[pl.ds(rbase, _RPT)], acc.at[pl.ds(rbase, _RPT)])
        pltpu.sync_copy(table.at[cid, pl.ds(rbase, _RPT)], tsh.at[pl.ds(rbase, _RPT)])
        plsc.subcore_barrier()
        _edge_pipeline(tsh, gidx3, sidx3, gib, sib, rowsb, acc,
                       sxs, sgs, sss, n_chunks, cb0, cid)
        _idx_wait(gidx3, sidx3, gib, sib, sxs, cb0, 0, 0, cid)
        plsc.subcore_barrier()
        pltpu.sync_copy(acc.at[pl.ds(rbase, _RPT)], out.at[cid, pl.ds(rbase, _RPT)])

    return pass_kernel


def _make_sc_appnp(interpret=False):
    fh = 32
    nch = _NCH_G
    nblk = _RPT // _CHUNK

    @functools.partial(
        pl.kernel,
        out_type=jax.ShapeDtypeStruct((2, _NP, fh), jnp.float32),
        mesh=_sc_mesh(),
        scratch_types=[
            pltpu.VMEM((2, _SUP, _CHUNK), jnp.int32),
            pltpu.VMEM((2, _SUP, _CHUNK), jnp.int32),
            pltpu.VMEM((4, _CHUNK, fh), jnp.float32),
            pltpu.VMEM((_RPT, fh), jnp.float32),
            pltpu.VMEM((_CHUNK, fh), jnp.float32),
            pltpu.VMEM((_CHUNK, 16), jnp.float32),
            pltpu.VMEM((_CHUNK, fh), jnp.float32),
            pltpu.VMEM((_CHUNK, fh), jnp.float32),
            pltpu.VMEM_SHARED((_NP, fh), jnp.float32),
            pltpu.VMEM_SHARED((_NP, fh), jnp.float32),
            pltpu.SemaphoreType.DMA,
            pltpu.SemaphoreType.DMA,
            pltpu.SemaphoreType.DMA,
            pltpu.SemaphoreType.DMA,
            pltpu.SemaphoreType.DMA,
            pltpu.SemaphoreType.DMA,
            pltpu.SemaphoreType.DMA,
            pltpu.SemaphoreType.DMA,
            pltpu.SemaphoreType.DMA,
            pltpu.SemaphoreType.DMA,
        ],
        compiler_params=pltpu.CompilerParams(use_tc_tiling_on_sc=False),
        interpret=interpret,
    )
    def appnp_kernel(th, evs, dv16, gidx3, sidx3, zrows, outT,
                     gib, sib, rowsb, tbuf, evb, dvb, abuf, zbuf, acc, tsh,
                     sx0, sx1, sg0, sg1, sg2, sg3, ss0, ss1, ss2, ss3):
        cid = lax.axis_index("c")
        sid = lax.axis_index("s")
        sxs = (sx0, sx1)
        sgs = (sg0, sg1, sg2, sg3)
        sss = (ss0, ss1, ss2, ss3)
        rbase = sid * _RPT
        cb0 = sid * nch
        _idx_start(gidx3, sidx3, gib, sib, sxs, cb0, 0, 0, cid)
        pltpu.sync_copy(zrows.at[pl.ds(rbase, _RPT)], acc.at[pl.ds(rbase, _RPT)])
        pltpu.sync_copy(th.at[cid, pl.ds(rbase, _RPT)], tbuf)
        pltpu.sync_copy(zrows.at[pl.ds(0, _CHUNK)], zbuf)
        pltpu.sync_copy(tbuf, tsh.at[pl.ds(rbase, _RPT)])

        def iter_body(k, carry):
            plsc.subcore_barrier()
            _edge_pipeline(tsh, gidx3, sidx3, gib, sib, rowsb, acc,
                           sxs, sgs, sss, nch, cb0, cid)
            plsc.subcore_barrier()
            for blk in range(nblk):
                row0 = rbase + blk * _CHUNK
                pltpu.sync_copy(acc.at[pl.ds(row0, _CHUNK)], abuf)
                pltpu.sync_copy(evs.at[cid, pl.ds(row0, _CHUNK)], evb)
                pltpu.sync_copy(dv16.at[pl.ds(row0, _CHUNK)], dvb)

                def grp_body(g, c2, _blk=blk):
                    for u in range(16):
                        la = g * 16 + u
                        lr = _blk * _CHUNK + la
                        dv = dvb[la, :]
                        for c in range(fh // 16):
                            sl = pl.ds(c * 16, 16)
                            tbuf[lr, sl] = dv * (abuf[la, sl] + tbuf[lr, sl]) + evb[la, sl]
                    return c2

                lax.fori_loop(0, _CHUNK // 16, grp_body, 0)
                pltpu.sync_copy(zbuf, acc.at[pl.ds(row0, _CHUNK)])
            pltpu.sync_copy(tbuf, tsh.at[pl.ds(rbase, _RPT)])
            return carry

        lax.fori_loop(0, _APPNP_K, iter_body, 0)
        _idx_wait(gidx3, sidx3, gib, sib, sxs, cb0, 0, 0, cid)
        pltpu.sync_copy(tbuf, outT.at[cid, pl.ds(rbase, _RPT)])

    return appnp_kernel


def _make_sc_degree(interpret=False):
    nsup = _NCH_D // _SUP

    @functools.partial(
        pl.kernel,
        out_type=jax.ShapeDtypeStruct((2, _NP, 16), jnp.float32),
        mesh=_sc_mesh(),
        scratch_types=[
            pltpu.VMEM((_SUP, _CHUNK), jnp.int32),
            pltpu.VMEM((_SUP, _CHUNK), jnp.int32),
            pltpu.VMEM((_CHUNK, 16), jnp.float32),
            pltpu.VMEM_SHARED((_NP, 16), jnp.float32),
            pltpu.SemaphoreType.DMA,
            pltpu.SemaphoreType.DMA,
            pltpu.SemaphoreType.DMA,
        ],
        compiler_params=pltpu.CompilerParams(use_tc_tiling_on_sc=False),
        interpret=interpret,
    )
    def deg_kernel(didx3, ones_hbm, zrows, out, si0, si1, ones_v, acc, sx0, sx1, ss):
        cid = lax.axis_index("c")
        sid = lax.axis_index("s")
        sis = (si0, si1)
        sxs = (sx0, sx1)
        rbase = sid * _RPT
        pltpu.sync_copy(zrows.at[pl.ds(rbase, _RPT)], acc.at[pl.ds(rbase, _RPT)])
        pltpu.sync_copy(ones_hbm, ones_v)
        plsc.subcore_barrier()
        wid = cid * 16 + sid
        cb0 = wid * _NCH_D

        def idx_start(s, b):
            pltpu.async_copy(didx3.at[pl.ds(cb0 + s * _SUP, _SUP)], sis[b], sxs[b])

        def idx_wait(s, b):
            pltpu.make_async_copy(
                didx3.at[pl.ds(cb0 + s * _SUP, _SUP)], sis[b], sxs[b]).wait()

        idx_start(0, 0)

        def sup_body(t, carry):
            for ph in range(2):
                s = 2 * t + ph
                idx_wait(s, ph)
                s_next = jnp.where(s + 1 == nsup, 0, s + 1)
                idx_start(s_next, 1 - ph)
                descs = [
                    pltpu.async_copy(ones_v, acc.at[sis[ph].at[j]], ss, add=True)
                    for j in range(_SUP)
                ]
                for d in descs:
                    d.wait()
            return carry

        lax.fori_loop(0, nsup // 2, sup_body, 0)
        idx_wait(0, 0)
        plsc.subcore_barrier()
        pltpu.sync_copy(acc.at[pl.ds(rbase, _RPT)], out.at[cid, pl.ds(rbase, _RPT)])

    return deg_kernel


_R = 1000


def _tc_call(body, out_shapes, in_specs, out_specs, interpret=False):
    return pl.pallas_call(
        body,
        out_shape=out_shapes,
        grid=(_N // _R,),
        in_specs=in_specs,
        out_specs=out_specs,
        interpret=interpret,
    )


def _spec_rows(width):
    return pl.BlockSpec((_R, width), lambda i: (i, 0))


def _spec_halves(width):
    return pl.BlockSpec((2, _R, width), lambda i: (0, i, 0))


def _spec_full(a, b):
    return pl.BlockSpec((a, b), lambda i: (0, 0))


def _tc1(x, w1, degp, interpret=False):
    def body(x_ref, w_ref, d_ref, dinv_ref, t_ref):
        deg = d_ref[0, :, 0] + d_ref[1, :, 0] + 1.0
        dinv = lax.rsqrt(deg)[:, None]
        dinv_ref[...] = dinv
        y = jnp.dot(x_ref[...], w_ref[...], preferred_element_type=jnp.float32)
        t = y * dinv
        t_ref[0] = t[:, :64]
        t_ref[1] = t[:, 64:]

    return _tc_call(
        body,
        [jax.ShapeDtypeStruct((_N, 1), jnp.float32),
         jax.ShapeDtypeStruct((2, _NP, 64), jnp.float32)],
        [_spec_rows(_H), _spec_full(_H, _H), _spec_halves(16)],
        [pl.BlockSpec((_R, 1), lambda i: (i, 0)), _spec_halves(64)],
        interpret,
    )(x, w1, degp)


def _tc2(agg, t0h, b1, dinv, interpret=False):
    def body(a_ref, t_ref, b_ref, dinv_ref, h_ref):
        d = dinv_ref[...]
        for c in range(2):
            v = d * (a_ref[c] + t_ref[c]) + b_ref[0, c * 64:(c + 1) * 64]
            h_ref[c] = jnp.maximum(v, 0.0)

    return _tc_call(
        body,
        jax.ShapeDtypeStruct((2, _NP, 64), jnp.float32),
        [_spec_halves(64), _spec_halves(64), _spec_full(1, _H),
         pl.BlockSpec((_R, 1), lambda i: (i, 0))],
        _spec_halves(64),
        interpret,
    )(agg, t0h, b1, dinv)


def _tc3(crf, h1h, w2, dinv, interpret=False):
    def body(c_ref, h_ref, w_ref, dinv_ref, t_ref):
        h1 = jnp.concatenate([h_ref[0], h_ref[1]], axis=1)
        cr = jnp.concatenate([c_ref[0], c_ref[1]], axis=1)
        hc = (1.0 - _CRF_ALPHA) * h1 + _CRF_ALPHA * cr
        t = dinv_ref[...] * jnp.dot(hc, w_ref[...], preferred_element_type=jnp.float32)
        t_ref[0] = t[:, :64]
        t_ref[1] = t[:, 64:]

    return _tc_call(
        body,
        jax.ShapeDtypeStruct((2, _NP, 64), jnp.float32),
        [_spec_halves(64), _spec_halves(64), _spec_full(_H, _H),
         pl.BlockSpec((_R, 1), lambda i: (i, 0))],
        _spec_halves(64),
        interpret,
    )(crf, h1h, w2, dinv)


def _tc4(agg, t1h, b2, wc, bc, we, be, dinv, interpret=False):
    def body(a_ref, t_ref, b2_ref, wc_ref, bc_ref, we_ref, be_ref, dinv_ref,
             th_ref, evs_ref, dv_ref):
        d = dinv_ref[...]
        hs = []
        for c in range(2):
            v = d * (a_ref[c] + t_ref[c]) + b2_ref[0, c * 64:(c + 1) * 64]
            hs.append(jnp.maximum(v, 0.0))
        h2 = jnp.concatenate(hs, axis=1)
        logits = jnp.dot(h2, wc_ref[...], preferred_element_type=jnp.float32) + bc_ref[0]
        ev = jnp.maximum(
            jnp.dot(logits, we_ref[...], preferred_element_type=jnp.float32) + be_ref[0],
            0.0)
        t = d * ev
        th_ref[0] = t[:, :32]
        th_ref[1] = t[:, 32:]
        evs = _APPNP_ALPHA * t
        evs_ref[0] = evs[:, :32]
        evs_ref[1] = evs[:, 32:]
        dv_ref[...] = jnp.broadcast_to((1.0 - _APPNP_ALPHA) * d * d, (_R, 16))

    return _tc_call(
        body,
        [jax.ShapeDtypeStruct((2, _NP, 32), jnp.float32),
         jax.ShapeDtypeStruct((2, _NP, 32), jnp.float32),
         jax.ShapeDtypeStruct((_NP, 16), jnp.float32)],
        [_spec_halves(64), _spec_halves(64), _spec_full(1, _H),
         _spec_full(_H, _C), _spec_full(1, _C), _spec_full(_C, _C),
         _spec_full(1, _C), pl.BlockSpec((_R, 1), lambda i: (i, 0))],
        [_spec_halves(32), _spec_halves(32), pl.BlockSpec((_R, 16), lambda i: (i, 0))],
        interpret,
    )(agg, t1h, b2, wc, bc, we, be, dinv)


def _tc6(tk, dinv, interpret=False):
    def body(t_ref, dinv_ref, o_ref):
        d = dinv_ref[...]
        h = jnp.concatenate([t_ref[0], t_ref[1]], axis=1) / d
        m = jnp.max(h, axis=1, keepdims=True)
        z = h - m
        lse = jnp.log(jnp.sum(jnp.exp(z), axis=1, keepdims=True))
        o_ref[...] = z - lse

    return _tc_call(
        body,
        jax.ShapeDtypeStruct((_N, _C), jnp.float32),
        [_spec_halves(32), pl.BlockSpec((_R, 1), lambda i: (i, 0))],
        _spec_rows(_C),
        interpret,
    )(tk, dinv)


def _run(x, edge_index, W1, b1, W2, b2, Wc, bc, We, be, interpret=False):
    src = edge_index[0]
    dst = edge_index[1]
    i32 = jnp.int32

    pad_g = _EP_G - _E
    gidx_g = jnp.concatenate([src, jnp.zeros((pad_g,), i32)])
    sidx_g = jnp.concatenate([dst, jnp.full((pad_g,), _N, i32)])
    pad_c = _EP_C - 2 * _E
    gidx_c = jnp.concatenate([dst, src, jnp.zeros((pad_c,), i32)])
    sidx_c = jnp.concatenate([src, dst, jnp.full((pad_c,), _N, i32)])
    pad_d = _EP_D - _E
    didx = jnp.concatenate([dst, jnp.full((pad_d,), _N, i32)])

    gidx_a = jnp.stack([gidx_g, gidx_g]).reshape(2, _EP_G // _CHUNK, _CHUNK)
    gidx_g = jnp.stack([gidx_g, gidx_g]).reshape(2, _EP_G // _CHUNK, _CHUNK)
    sidx_g = sidx_g.reshape(_EP_G // _CHUNK, _CHUNK)
    gidx_c = jnp.stack([gidx_c, gidx_c]).reshape(2, _EP_C // _CHUNK, _CHUNK)
    sidx_c = sidx_c.reshape(_EP_C // _CHUNK, _CHUNK)
    didx = didx.reshape(_EP_D // _CHUNK, _CHUNK)

    z64 = jnp.zeros((_NP, 64), jnp.float32)
    z32 = jnp.zeros((_NP, 32), jnp.float32)
    z16 = jnp.zeros((_NP, 16), jnp.float32)
    on16 = jnp.ones((_CHUNK, 16), jnp.float32)

    b1r = b1.reshape(1, _H)
    b2r = b2.reshape(1, _H)
    bcr = bc.reshape(1, _C)
    ber = be.reshape(1, _C)

    pass64 = _make_sc_pass(64, _NCH_G, interpret)
    pass64c = _make_sc_pass(64, _NCH_C, interpret)

    degp = _make_sc_degree(interpret)(didx, on16, z16)
    dinv, t0h = _tc1(x, W1, degp, interpret)

    agg1 = pass64(t0h, gidx_g, sidx_g, z64)
    h1h = _tc2(agg1, t0h, b1r, dinv, interpret)

    crf = pass64c(h1h, gidx_c, sidx_c, z64)
    t1h = _tc3(crf, h1h, W2, dinv, interpret)

    agg2 = pass64(t1h, gidx_g, sidx_g, z64)
    th, evs, dv16 = _tc4(agg2, t1h, b2r, Wc, bcr, We, ber, dinv, interpret)

    tk = _make_sc_appnp(interpret)(th, evs, dv16, gidx_a, sidx_g, z32)
    return _tc6(tk, dinv, interpret)


def kernel(x, edge_index, W1, b1, W2, b2, Wc, bc, We, be):
    return _run(x, edge_index, W1, b1, W2, b2, Wc, bc, We, be)
